# baseline (speedup 1.0000x reference)
import functools

import jax
import jax.numpy as jnp
from jax import lax
from jax.experimental import pallas as pl
from jax.experimental.pallas import tpu as pltpu
from jax.experimental.pallas import tpu_sc as plsc

N = 10000
E = 320000
D = 128
G = 128
OUT = 128

NP = 10240
NW = 32
KC = 80
TOTC = 4096
EPAD = TOTC * KC
GC = 8
AC0 = 48
AC1 = 256 - AC0
CH_SC = TOTC // 16
ROWS_T = NP // 16
BM = 1024

_mesh = plsc.VectorSubcoreMesh(core_axis_name="c", subcore_axis_name="s")
_sc_params = pltpu.CompilerParams(needs_layout_passes=False)


def _rsqrt16(x):
    s = 0.5 * (1.0 + x)
    for _ in range(15):
        s = 0.5 * (s + x / s)
    return 1.0 / s


def _zero_fill(buf, nrows):
    def body(i, _):
        for j in range(8):
            buf[i, pl.ds(j * 16, 16)] = jnp.zeros((16,), jnp.float32)
        return 0
    lax.fori_loop(0, nrows, body, 0)


NSEM = 8


def _fire_drain(nchunks, fire):
    full = nchunks // NSEM
    rem = nchunks - full * NSEM

    def grp(g, _):
        base = g * NSEM
        ds_ = [fire(base + k, k) for k in range(NSEM)]
        for dsc in ds_:
            dsc.wait()
        return 0
    lax.fori_loop(0, full, grp, 0)
    ds_ = [fire(full * NSEM + k, k) for k in range(rem)]
    for dsc in ds_:
        dsc.wait()


@functools.partial(
    pl.kernel,
    out_type=(jax.ShapeDtypeStruct((NP,), jnp.float32),
              jax.ShapeDtypeStruct((NP,), jnp.float32)),
    mesh=_mesh,
    compiler_params=_sc_params,
    scratch_types=dict(
        deg_acc=pltpu.VMEM_SHARED((NP,), jnp.float32),
        cs_acc=pltpu.VMEM_SHARED((NP,), jnp.float32),
        dinv_sh=pltpu.VMEM_SHARED((NP,), jnp.float32),
        onesv=pltpu.VMEM((KC,), jnp.float32),
        srcv=pltpu.VMEM((CH_SC, KC), jnp.int32),
        dstv=pltpu.VMEM((CH_SC, KC), jnp.int32),
        valv=pltpu.VMEM((CH_SC, KC), jnp.float32),
        dv=pltpu.VMEM((NP,), jnp.float32),
        dslice=pltpu.VMEM((ROWS_T,), jnp.float32),
        csv=pltpu.VMEM((ROWS_T,), jnp.float32),
        sems=pltpu.SemaphoreType.DMA((NSEM,)),
    ),
)
def _sc_scalar(src_hbm, dst_hbm, dinv_out, cc_out, *, deg_acc, cs_acc,
               dinv_sh, onesv, srcv, dstv, valv, dv, dslice, csv, sems):
    c = lax.axis_index("c")
    s = lax.axis_index("s")

    @pl.when(c == 0)
    def _():
        r0 = s * ROWS_T
        def zb(i, _):
            dslice[pl.ds(i * 16, 16)] = jnp.zeros((16,), jnp.float32)
            return 0
        lax.fori_loop(0, ROWS_T // 16, zb, 0)
        pltpu.sync_copy(dslice, deg_acc.at[pl.ds(r0, ROWS_T)])
        pltpu.sync_copy(dslice, cs_acc.at[pl.ds(r0, ROWS_T)])

        def ob(i, _):
            onesv[pl.ds(i * 16, 16)] = jnp.ones((16,), jnp.float32)
            return 0
        lax.fori_loop(0, KC // 16, ob, 0)
        plsc.subcore_barrier()

        pltpu.sync_copy(dst_hbm.at[pl.ds(s * CH_SC, CH_SC)], dstv)

        def fire_deg(i, k):
            return pltpu.async_copy(
                onesv, deg_acc.at[dstv.at[i]], sems.at[k], add=True)
        _fire_drain(CH_SC, fire_deg)
        plsc.subcore_barrier()

        pltpu.sync_copy(deg_acc.at[pl.ds(r0, ROWS_T)], csv)

        def rb(i, _):
            x = csv[pl.ds(i * 16, 16)] + 1.0
            dslice[pl.ds(i * 16, 16)] = _rsqrt16(x)
            return 0
        lax.fori_loop(0, ROWS_T // 16, rb, 0)
        pltpu.sync_copy(dslice, dinv_sh.at[pl.ds(r0, ROWS_T)])
        plsc.subcore_barrier()

        pltpu.sync_copy(dinv_sh, dv)
        pltpu.sync_copy(src_hbm.at[pl.ds(s * CH_SC, CH_SC)], srcv)

        def gb(i, _):
            for j in range(KC // 16):
                idx = srcv[i, pl.ds(j * 16, 16)]
                valv[i, pl.ds(j * 16, 16)] = plsc.load_gather(dv, [idx])
            return 0
        lax.fori_loop(0, CH_SC, gb, 0)

        def fire_cs(i, k):
            return pltpu.async_copy(
                valv.at[i], cs_acc.at[dstv.at[i]], sems.at[k], add=True)
        _fire_drain(CH_SC, fire_cs)
        plsc.subcore_barrier()

        pltpu.sync_copy(cs_acc.at[pl.ds(r0, ROWS_T)], csv)

        def cb(i, _):
            dvv = dslice[pl.ds(i * 16, 16)]
            csv[pl.ds(i * 16, 16)] = dvv * (csv[pl.ds(i * 16, 16)] + dvv)
            return 0
        lax.fori_loop(0, ROWS_T // 16, cb, 0)
        pltpu.sync_copy(dslice, dinv_out.at[pl.ds(r0, ROWS_T)])
        pltpu.sync_copy(csv, cc_out.at[pl.ds(r0, ROWS_T)])


RING = 4
NAGG = 10112
ROWS_A = NAGG // 16


@functools.partial(
    pl.kernel,
    out_type=jax.ShapeDtypeStruct((2, NP, D), jnp.float32),
    mesh=_mesh,
    compiler_params=_sc_params,
    scratch_types=dict(
        acc=pltpu.VMEM_SHARED((NAGG, D), jnp.float32),
        sgrp=pltpu.VMEM((2, GC, KC), jnp.int32),
        dgrp=pltpu.VMEM((2, GC, KC), jnp.int32),
        rowbuf=pltpu.VMEM((RING, KC, D), jnp.float32),
        gsems=pltpu.SemaphoreType.DMA((RING,)),
        ssems=pltpu.SemaphoreType.DMA((RING,)),
        isems=pltpu.SemaphoreType.DMA((2,)),
    ),
)
def _sc_agg(gsc_hbm, src_hbm, dst_hbm, out_hbm, *, acc, sgrp, dgrp, rowbuf,
            gsems, ssems, isems):
    c = lax.axis_index("c")
    s = lax.axis_index("s")
    r0 = s * ROWS_A
    nc = jnp.where(c == 0, AC0, AC1)
    base = jnp.where(c == 0, s * AC0, 16 * AC0 + s * AC1)

    _zero_fill(rowbuf.at[0], KC)
    for k in range(ROWS_A // KC):
        pltpu.sync_copy(rowbuf.at[0], acc.at[pl.ds(r0 + k * KC, KC)])
    rem_rows = ROWS_A - (ROWS_A // KC) * KC
    pltpu.sync_copy(rowbuf.at[0, pl.ds(0, rem_rows)],
                    acc.at[pl.ds(r0 + (ROWS_A // KC) * KC, rem_rows)])
    plsc.subcore_barrier()

    pltpu.sync_copy(src_hbm.at[pl.ds(base, GC)], sgrp.at[0])
    pltpu.sync_copy(dst_hbm.at[pl.ds(base, GC)], dgrp.at[0])
    pltpu.async_copy(gsc_hbm.at[sgrp.at[0, 0]], rowbuf.at[0], gsems.at[0])
    pltpu.async_copy(gsc_hbm.at[sgrp.at[0, 1]], rowbuf.at[1], gsems.at[1])

    def it(i, _):
        g = lax.div(i, GC)
        j = lax.rem(i, GC)
        p = lax.rem(g, 2)
        b = lax.rem(i, RING)
        b2 = lax.rem(i + 2, RING)
        nb = base + (g + 1) * GC
        have_next_grp = (g + 1) * GC < nc

        @pl.when(jnp.logical_and(i >= 2, i <= nc - 3))
        def _():
            pltpu.make_async_copy(rowbuf.at[b2], acc.at[dgrp.at[0, 0]],
                                  ssems.at[b2]).wait()

        @pl.when(jnp.logical_and(j == 1, have_next_grp))
        def _():
            pltpu.async_copy(src_hbm.at[pl.ds(nb, GC)], sgrp.at[1 - p],
                             isems.at[0])
            pltpu.async_copy(dst_hbm.at[pl.ds(nb, GC)], dgrp.at[1 - p],
                             isems.at[1])

        @pl.when(jnp.logical_and(j == GC - 2, have_next_grp))
        def _():
            pltpu.make_async_copy(src_hbm.at[pl.ds(nb, GC)], sgrp.at[1 - p],
                                  isems.at[0]).wait()
            pltpu.make_async_copy(dst_hbm.at[pl.ds(nb, GC)], dgrp.at[1 - p],
                                  isems.at[1]).wait()

        @pl.when(i <= nc - 3)
        def _():
            g2 = lax.div(i + 2, GC)
            j2 = lax.rem(i + 2, GC)
            p2 = lax.rem(g2, 2)
            pltpu.async_copy(gsc_hbm.at[sgrp.at[p2, j2]], rowbuf.at[b2],
                             gsems.at[b2])

        pltpu.make_async_copy(gsc_hbm.at[sgrp.at[p, j]], rowbuf.at[b],
                              gsems.at[b]).wait()
        pltpu.async_copy(rowbuf.at[b], acc.at[dgrp.at[p, j]], ssems.at[b],
                         add=True)
        return 0

    lax.fori_loop(0, nc, it, 0)
    for v in range(RING):
        pltpu.make_async_copy(rowbuf.at[v], acc.at[dgrp.at[0, 0]],
                              ssems.at[v]).wait()

    plsc.subcore_barrier()
    pltpu.sync_copy(acc.at[pl.ds(r0, ROWS_A)],
                    out_hbm.at[c, pl.ds(r0, ROWS_A)])


def _elu(z):
    return jnp.where(z > 0, z, jnp.exp(z) - 1.0)


def _tc_a_body(emb_r, w1_r, b1_r, w2_r, dinv_r, cc_r, h1_r, g2_r):
    e = emb_r[...]
    r1 = jnp.dot(e, w1_r[...], preferred_element_type=jnp.float32)
    h1 = _elu(cc_r[...] * r1 + b1_r[...])
    h1_r[...] = h1
    x2 = h1 + e
    g2_r[...] = dinv_r[...] * jnp.dot(x2, w2_r[...],
                                      preferred_element_type=jnp.float32)


def _tc_a(emb, W1, b1, W2, dinv_c, cc_c):
    grid = (NP // BM,)
    cst = lambda i: (0, 0)
    row = lambda i: (i, 0)
    return pl.pallas_call(
        _tc_a_body,
        grid=grid,
        in_specs=[
            pl.BlockSpec((1, D), cst),
            pl.BlockSpec((D, D), cst),
            pl.BlockSpec((1, D), cst),
            pl.BlockSpec((D, D), cst),
            pl.BlockSpec((BM, 1), row),
            pl.BlockSpec((BM, 1), row),
        ],
        out_specs=[pl.BlockSpec((BM, D), row), pl.BlockSpec((BM, D), row)],
        out_shape=[jax.ShapeDtypeStruct((NP, D), jnp.float32),
                   jax.ShapeDtypeStruct((NP, D), jnp.float32)],
    )(emb, W1, b1, W2, dinv_c, cc_c)


def _tc_b_body(p_r, g2_r, h1_r, dinv_r, emb_r, w3_r, b2_r, h2_r, g3_r):
    agg = dinv_r[...] * (p_r[0] + p_r[1] + g2_r[...])
    h2 = _elu(agg + b2_r[...])
    h2_r[...] = h2
    x3 = emb_r[...] + h1_r[...] + h2
    g3_r[...] = dinv_r[...] * jnp.dot(x3, w3_r[...],
                                      preferred_element_type=jnp.float32)


def _tc_b(p, g2, h1, dinv_c, emb, W3, b2):
    grid = (NP // BM,)
    cst = lambda i: (0, 0)
    row = lambda i: (i, 0)
    return pl.pallas_call(
        _tc_b_body,
        grid=grid,
        in_specs=[
            pl.BlockSpec((2, BM, D), lambda i: (0, i, 0)),
            pl.BlockSpec((BM, D), row),
            pl.BlockSpec((BM, D), row),
            pl.BlockSpec((BM, 1), row),
            pl.BlockSpec((1, D), cst),
            pl.BlockSpec((D, D), cst),
            pl.BlockSpec((1, D), cst),
        ],
        out_specs=[pl.BlockSpec((BM, D), row), pl.BlockSpec((BM, D), row)],
        out_shape=[jax.ShapeDtypeStruct((NP, D), jnp.float32),
                   jax.ShapeDtypeStruct((NP, D), jnp.float32)],
    )(p, g2, h1, dinv_c, emb, W3, b2)


def _tc_c_body(p_r, g3_r, h1_r, h2_r, dinv_r, emb_r, b3_r, lw_r, lb_r, pw_r,
               pb_r, batch_r, out_r, pooled):
    i = pl.program_id(0)
    h3 = _elu(dinv_r[...] * (p_r[0] + p_r[1] + g3_r[...]) + b3_r[...])
    xf = emb_r[...] + h1_r[...] + h2_r[...] + h3
    hf = _elu(jnp.dot(xf, lw_r[...], preferred_element_type=jnp.float32)
              + lb_r[...])
    rid = i * BM + lax.broadcasted_iota(jnp.int32, (BM, 1), 0)
    hf = jnp.where(rid < N, hf, 0.0)
    oh = (batch_r[...] == lax.broadcasted_iota(jnp.int32, (BM, G), 1)
          ).astype(jnp.float32)
    contrib = lax.dot_general(oh, hf, (((0,), (0,)), ((), ())),
                              preferred_element_type=jnp.float32)

    @pl.when(i == 0)
    def _():
        pooled[...] = jnp.zeros((G, D), jnp.float32)

    pooled[...] += contrib
    out_r[...] = (jnp.dot(pooled[...], pw_r[...],
                          preferred_element_type=jnp.float32)
                  + pb_r[...]) * 0.1


def _tc_c(p, g3, h1, h2, dinv_c, emb, b3, last_W, last_b, pred_W, pred_b,
          batch_c):
    grid = (NP // BM,)
    cst = lambda i: (0, 0)
    row = lambda i: (i, 0)
    return pl.pallas_call(
        _tc_c_body,
        grid=grid,
        in_specs=[
            pl.BlockSpec((2, BM, D), lambda i: (0, i, 0)),
            pl.BlockSpec((BM, D), row),
            pl.BlockSpec((BM, D), row),
            pl.BlockSpec((BM, D), row),
            pl.BlockSpec((BM, 1), row),
            pl.BlockSpec((1, D), cst),
            pl.BlockSpec((1, D), cst),
            pl.BlockSpec((D, D), cst),
            pl.BlockSpec((1, D), cst),
            pl.BlockSpec((D, OUT), cst),
            pl.BlockSpec((1, OUT), cst),
            pl.BlockSpec((BM, 1), row),
        ],
        out_specs=pl.BlockSpec((G, OUT), cst),
        out_shape=jax.ShapeDtypeStruct((G, OUT), jnp.float32),
        scratch_shapes=[pltpu.VMEM((G, D), jnp.float32)],
    )(p, g3, h1, h2, dinv_c, emb, b3, last_W, last_b, pred_W, pred_b, batch_c)


def kernel(x, edge_index, batch, edge_attr, emb, W1, b1, W2, b2, W3, b3,
           last_W, last_b, pred_W, pred_b):
    src = edge_index[0]
    dst = edge_index[1]
    srcp = jnp.concatenate(
        [src, jnp.zeros((EPAD - E,), jnp.int32)]).reshape(TOTC, KC)
    dstp = jnp.concatenate(
        [dst, jnp.full((EPAD - E,), N, jnp.int32)]).reshape(TOTC, KC)
    batch_c = jnp.concatenate(
        [batch, jnp.full((NP - N,), G, jnp.int32)]).reshape(NP, 1)

    dinv, cc = _sc_scalar(srcp, dstp)
    dinv_c = dinv.reshape(NP, 1)
    cc_c = cc.reshape(NP, 1)

    h1, g2 = _tc_a(emb, W1, b1.reshape(1, D), W2, dinv_c, cc_c)
    p2 = _sc_agg(g2, srcp, dstp)
    h2, g3 = _tc_b(p2, g2, h1, dinv_c, emb, W3, b2.reshape(1, D))
    p3 = _sc_agg(g3, srcp, dstp)
    out = _tc_c(p3, g3, h1, h2, dinv_c, emb, b3.reshape(1, D), last_W,
                last_b.reshape(1, D), pred_W, pred_b.reshape(1, OUT), batch_c)
    return out


# speedup vs baseline: 1.1054x; 1.1054x over previous
import functools

import jax
import jax.numpy as jnp
from jax import lax
from jax.experimental import pallas as pl
from jax.experimental.pallas import tpu as pltpu
from jax.experimental.pallas import tpu_sc as plsc

N = 10000
E = 320000
D = 128
G = 128
OUT = 128

NP = 10240
NW = 32
KC = 80
TOTC = 4096
EPAD = TOTC * KC
GC = 8
AC0 = 208
AC1 = 256 - AC0
CH_SC = TOTC // 16
ROWS_T = NP // 16
BM = 1024

_mesh = plsc.VectorSubcoreMesh(core_axis_name="c", subcore_axis_name="s")
_sc_params = pltpu.CompilerParams(needs_layout_passes=False)


def _rsqrt16(x):
    s = 0.5 * (1.0 + x)
    for _ in range(15):
        s = 0.5 * (s + x / s)
    return 1.0 / s


def _zero_fill(buf, nrows):
    def body(i, _):
        for j in range(8):
            buf[i, pl.ds(j * 16, 16)] = jnp.zeros((16,), jnp.float32)
        return 0
    lax.fori_loop(0, nrows, body, 0)


NSEM = 8


def _fire_drain(nchunks, fire):
    full = nchunks // NSEM
    rem = nchunks - full * NSEM

    def grp(g, _):
        base = g * NSEM
        ds_ = [fire(base + k, k) for k in range(NSEM)]
        for dsc in ds_:
            dsc.wait()
        return 0
    lax.fori_loop(0, full, grp, 0)
    ds_ = [fire(full * NSEM + k, k) for k in range(rem)]
    for dsc in ds_:
        dsc.wait()


@functools.partial(
    pl.kernel,
    out_type=(jax.ShapeDtypeStruct((NP,), jnp.float32),
              jax.ShapeDtypeStruct((NP,), jnp.float32)),
    mesh=_mesh,
    compiler_params=_sc_params,
    scratch_types=dict(
        deg_acc=pltpu.VMEM_SHARED((NP,), jnp.float32),
        cs_acc=pltpu.VMEM_SHARED((NP,), jnp.float32),
        dinv_sh=pltpu.VMEM_SHARED((NP,), jnp.float32),
        onesv=pltpu.VMEM((KC,), jnp.float32),
        srcv=pltpu.VMEM((CH_SC, KC), jnp.int32),
        dstv=pltpu.VMEM((CH_SC, KC), jnp.int32),
        valv=pltpu.VMEM((CH_SC, KC), jnp.float32),
        dv=pltpu.VMEM((NP,), jnp.float32),
        dslice=pltpu.VMEM((ROWS_T,), jnp.float32),
        csv=pltpu.VMEM((ROWS_T,), jnp.float32),
        sems=pltpu.SemaphoreType.DMA((NSEM,)),
    ),
)
def _sc_scalar(src_hbm, dst_hbm, dinv_out, cc_out, *, deg_acc, cs_acc,
               dinv_sh, onesv, srcv, dstv, valv, dv, dslice, csv, sems):
    c = lax.axis_index("c")
    s = lax.axis_index("s")

    @pl.when(c == 0)
    def _():
        r0 = s * ROWS_T
        def zb(i, _):
            dslice[pl.ds(i * 16, 16)] = jnp.zeros((16,), jnp.float32)
            return 0
        lax.fori_loop(0, ROWS_T // 16, zb, 0)
        pltpu.sync_copy(dslice, deg_acc.at[pl.ds(r0, ROWS_T)])
        pltpu.sync_copy(dslice, cs_acc.at[pl.ds(r0, ROWS_T)])

        def ob(i, _):
            onesv[pl.ds(i * 16, 16)] = jnp.ones((16,), jnp.float32)
            return 0
        lax.fori_loop(0, KC // 16, ob, 0)
        plsc.subcore_barrier()

        pltpu.sync_copy(dst_hbm.at[pl.ds(s * CH_SC, CH_SC)], dstv)

        def fire_deg(i, k):
            return pltpu.async_copy(
                onesv, deg_acc.at[dstv.at[i]], sems.at[k], add=True)
        _fire_drain(CH_SC, fire_deg)
        plsc.subcore_barrier()

        pltpu.sync_copy(deg_acc.at[pl.ds(r0, ROWS_T)], csv)

        def rb(i, _):
            x = csv[pl.ds(i * 16, 16)] + 1.0
            dslice[pl.ds(i * 16, 16)] = _rsqrt16(x)
            return 0
        lax.fori_loop(0, ROWS_T // 16, rb, 0)
        pltpu.sync_copy(dslice, dinv_sh.at[pl.ds(r0, ROWS_T)])
        plsc.subcore_barrier()

        pltpu.sync_copy(dinv_sh, dv)
        pltpu.sync_copy(src_hbm.at[pl.ds(s * CH_SC, CH_SC)], srcv)

        def gb(i, _):
            for j in range(KC // 16):
                idx = srcv[i, pl.ds(j * 16, 16)]
                valv[i, pl.ds(j * 16, 16)] = plsc.load_gather(dv, [idx])
            return 0
        lax.fori_loop(0, CH_SC, gb, 0)

        def fire_cs(i, k):
            return pltpu.async_copy(
                valv.at[i], cs_acc.at[dstv.at[i]], sems.at[k], add=True)
        _fire_drain(CH_SC, fire_cs)
        plsc.subcore_barrier()

        pltpu.sync_copy(cs_acc.at[pl.ds(r0, ROWS_T)], csv)

        def cb(i, _):
            dvv = dslice[pl.ds(i * 16, 16)]
            csv[pl.ds(i * 16, 16)] = dvv * (csv[pl.ds(i * 16, 16)] + dvv)
            return 0
        lax.fori_loop(0, ROWS_T // 16, cb, 0)
        pltpu.sync_copy(dslice, dinv_out.at[pl.ds(r0, ROWS_T)])
        pltpu.sync_copy(csv, cc_out.at[pl.ds(r0, ROWS_T)])


RING = 4
NAGG = 10112
ROWS_A = NAGG // 16


@functools.partial(
    pl.kernel,
    out_type=jax.ShapeDtypeStruct((2, NP, D), jnp.float32),
    mesh=_mesh,
    compiler_params=_sc_params,
    scratch_types=dict(
        acc=pltpu.VMEM_SHARED((NAGG, D), jnp.float32),
        sgrp=pltpu.VMEM((2, GC, KC), jnp.int32),
        dgrp=pltpu.VMEM((2, GC, KC), jnp.int32),
        rowbuf=pltpu.VMEM((RING, KC, D), jnp.float32),
        gsems=pltpu.SemaphoreType.DMA((RING,)),
        ssems=pltpu.SemaphoreType.DMA((RING,)),
        isems=pltpu.SemaphoreType.DMA((2,)),
    ),
)
def _sc_agg(gsc_hbm, src_hbm, dst_hbm, out_hbm, *, acc, sgrp, dgrp, rowbuf,
            gsems, ssems, isems):
    c = lax.axis_index("c")
    s = lax.axis_index("s")
    r0 = s * ROWS_A
    nc = jnp.where(c == 0, AC0, AC1)
    base = jnp.where(c == 0, s * AC0, 16 * AC0 + s * AC1)

    _zero_fill(rowbuf.at[0], KC)
    for k in range(ROWS_A // KC):
        pltpu.sync_copy(rowbuf.at[0], acc.at[pl.ds(r0 + k * KC, KC)])
    rem_rows = ROWS_A - (ROWS_A // KC) * KC
    pltpu.sync_copy(rowbuf.at[0, pl.ds(0, rem_rows)],
                    acc.at[pl.ds(r0 + (ROWS_A // KC) * KC, rem_rows)])
    plsc.subcore_barrier()

    pltpu.sync_copy(src_hbm.at[pl.ds(base, GC)], sgrp.at[0])
    pltpu.sync_copy(dst_hbm.at[pl.ds(base, GC)], dgrp.at[0])
    pltpu.async_copy(gsc_hbm.at[sgrp.at[0, 0]], rowbuf.at[0], gsems.at[0])
    pltpu.async_copy(gsc_hbm.at[sgrp.at[0, 1]], rowbuf.at[1], gsems.at[1])

    def it(i, _):
        g = lax.div(i, GC)
        j = lax.rem(i, GC)
        p = lax.rem(g, 2)
        b = lax.rem(i, RING)
        b2 = lax.rem(i + 2, RING)
        nb = base + (g + 1) * GC
        have_next_grp = (g + 1) * GC < nc

        @pl.when(jnp.logical_and(i >= 2, i <= nc - 3))
        def _():
            pltpu.make_async_copy(rowbuf.at[b2], acc.at[dgrp.at[0, 0]],
                                  ssems.at[b2]).wait()

        @pl.when(jnp.logical_and(j == 1, have_next_grp))
        def _():
            pltpu.async_copy(src_hbm.at[pl.ds(nb, GC)], sgrp.at[1 - p],
                             isems.at[0])
            pltpu.async_copy(dst_hbm.at[pl.ds(nb, GC)], dgrp.at[1 - p],
                             isems.at[1])

        @pl.when(jnp.logical_and(j == GC - 2, have_next_grp))
        def _():
            pltpu.make_async_copy(src_hbm.at[pl.ds(nb, GC)], sgrp.at[1 - p],
                                  isems.at[0]).wait()
            pltpu.make_async_copy(dst_hbm.at[pl.ds(nb, GC)], dgrp.at[1 - p],
                                  isems.at[1]).wait()

        @pl.when(i <= nc - 3)
        def _():
            g2 = lax.div(i + 2, GC)
            j2 = lax.rem(i + 2, GC)
            p2 = lax.rem(g2, 2)
            pltpu.async_copy(gsc_hbm.at[sgrp.at[p2, j2]], rowbuf.at[b2],
                             gsems.at[b2])

        pltpu.make_async_copy(gsc_hbm.at[sgrp.at[p, j]], rowbuf.at[b],
                              gsems.at[b]).wait()
        pltpu.async_copy(rowbuf.at[b], acc.at[dgrp.at[p, j]], ssems.at[b],
                         add=True)
        return 0

    lax.fori_loop(0, nc, it, 0)
    for v in range(RING):
        pltpu.make_async_copy(rowbuf.at[v], acc.at[dgrp.at[0, 0]],
                              ssems.at[v]).wait()

    plsc.subcore_barrier()
    pltpu.sync_copy(acc.at[pl.ds(r0, ROWS_A)],
                    out_hbm.at[c, pl.ds(r0, ROWS_A)])


def _elu(z):
    return jnp.where(z > 0, z, jnp.exp(z) - 1.0)


def _tc_a_body(emb_r, w1_r, b1_r, w2_r, dinv_r, cc_r, h1_r, g2_r):
    e = emb_r[...]
    r1 = jnp.dot(e, w1_r[...], preferred_element_type=jnp.float32)
    h1 = _elu(cc_r[...] * r1 + b1_r[...])
    h1_r[...] = h1
    x2 = h1 + e
    g2_r[...] = dinv_r[...] * jnp.dot(x2, w2_r[...],
                                      preferred_element_type=jnp.float32)


def _tc_a(emb, W1, b1, W2, dinv_c, cc_c):
    grid = (NP // BM,)
    cst = lambda i: (0, 0)
    row = lambda i: (i, 0)
    return pl.pallas_call(
        _tc_a_body,
        grid=grid,
        in_specs=[
            pl.BlockSpec((1, D), cst),
            pl.BlockSpec((D, D), cst),
            pl.BlockSpec((1, D), cst),
            pl.BlockSpec((D, D), cst),
            pl.BlockSpec((BM, 1), row),
            pl.BlockSpec((BM, 1), row),
        ],
        out_specs=[pl.BlockSpec((BM, D), row), pl.BlockSpec((BM, D), row)],
        out_shape=[jax.ShapeDtypeStruct((NP, D), jnp.float32),
                   jax.ShapeDtypeStruct((NP, D), jnp.float32)],
    )(emb, W1, b1, W2, dinv_c, cc_c)


def _tc_b_body(p_r, g2_r, h1_r, dinv_r, emb_r, w3_r, b2_r, h2_r, g3_r):
    agg = dinv_r[...] * (p_r[0] + p_r[1] + g2_r[...])
    h2 = _elu(agg + b2_r[...])
    h2_r[...] = h2
    x3 = emb_r[...] + h1_r[...] + h2
    g3_r[...] = dinv_r[...] * jnp.dot(x3, w3_r[...],
                                      preferred_element_type=jnp.float32)


def _tc_b(p, g2, h1, dinv_c, emb, W3, b2):
    grid = (NP // BM,)
    cst = lambda i: (0, 0)
    row = lambda i: (i, 0)
    return pl.pallas_call(
        _tc_b_body,
        grid=grid,
        in_specs=[
            pl.BlockSpec((2, BM, D), lambda i: (0, i, 0)),
            pl.BlockSpec((BM, D), row),
            pl.BlockSpec((BM, D), row),
            pl.BlockSpec((BM, 1), row),
            pl.BlockSpec((1, D), cst),
            pl.BlockSpec((D, D), cst),
            pl.BlockSpec((1, D), cst),
        ],
        out_specs=[pl.BlockSpec((BM, D), row), pl.BlockSpec((BM, D), row)],
        out_shape=[jax.ShapeDtypeStruct((NP, D), jnp.float32),
                   jax.ShapeDtypeStruct((NP, D), jnp.float32)],
    )(p, g2, h1, dinv_c, emb, W3, b2)


def _tc_c_body(p_r, g3_r, h1_r, h2_r, dinv_r, emb_r, b3_r, lw_r, lb_r, pw_r,
               pb_r, batch_r, out_r, pooled):
    i = pl.program_id(0)
    h3 = _elu(dinv_r[...] * (p_r[0] + p_r[1] + g3_r[...]) + b3_r[...])
    xf = emb_r[...] + h1_r[...] + h2_r[...] + h3
    hf = _elu(jnp.dot(xf, lw_r[...], preferred_element_type=jnp.float32)
              + lb_r[...])
    rid = i * BM + lax.broadcasted_iota(jnp.int32, (BM, 1), 0)
    hf = jnp.where(rid < N, hf, 0.0)
    oh = (batch_r[...] == lax.broadcasted_iota(jnp.int32, (BM, G), 1)
          ).astype(jnp.float32)
    contrib = lax.dot_general(oh, hf, (((0,), (0,)), ((), ())),
                              preferred_element_type=jnp.float32)

    @pl.when(i == 0)
    def _():
        pooled[...] = jnp.zeros((G, D), jnp.float32)

    pooled[...] += contrib
    out_r[...] = (jnp.dot(pooled[...], pw_r[...],
                          preferred_element_type=jnp.float32)
                  + pb_r[...]) * 0.1


def _tc_c(p, g3, h1, h2, dinv_c, emb, b3, last_W, last_b, pred_W, pred_b,
          batch_c):
    grid = (NP // BM,)
    cst = lambda i: (0, 0)
    row = lambda i: (i, 0)
    return pl.pallas_call(
        _tc_c_body,
        grid=grid,
        in_specs=[
            pl.BlockSpec((2, BM, D), lambda i: (0, i, 0)),
            pl.BlockSpec((BM, D), row),
            pl.BlockSpec((BM, D), row),
            pl.BlockSpec((BM, D), row),
            pl.BlockSpec((BM, 1), row),
            pl.BlockSpec((1, D), cst),
            pl.BlockSpec((1, D), cst),
            pl.BlockSpec((D, D), cst),
            pl.BlockSpec((1, D), cst),
            pl.BlockSpec((D, OUT), cst),
            pl.BlockSpec((1, OUT), cst),
            pl.BlockSpec((BM, 1), row),
        ],
        out_specs=pl.BlockSpec((G, OUT), cst),
        out_shape=jax.ShapeDtypeStruct((G, OUT), jnp.float32),
        scratch_shapes=[pltpu.VMEM((G, D), jnp.float32)],
    )(p, g3, h1, h2, dinv_c, emb, b3, last_W, last_b, pred_W, pred_b, batch_c)


def kernel(x, edge_index, batch, edge_attr, emb, W1, b1, W2, b2, W3, b3,
           last_W, last_b, pred_W, pred_b):
    src = edge_index[0]
    dst = edge_index[1]
    srcp = jnp.concatenate(
        [src, jnp.zeros((EPAD - E,), jnp.int32)]).reshape(TOTC, KC)
    dstp = jnp.concatenate(
        [dst, jnp.full((EPAD - E,), N, jnp.int32)]).reshape(TOTC, KC)
    batch_c = jnp.concatenate(
        [batch, jnp.full((NP - N,), G, jnp.int32)]).reshape(NP, 1)

    dinv, cc = _sc_scalar(srcp, dstp)
    dinv_c = dinv.reshape(NP, 1)
    cc_c = cc.reshape(NP, 1)

    h1, g2 = _tc_a(emb, W1, b1.reshape(1, D), W2, dinv_c, cc_c)
    p2 = _sc_agg(g2, srcp, dstp)
    h2, g3 = _tc_b(p2, g2, h1, dinv_c, emb, W3, b2.reshape(1, D))
    p3 = _sc_agg(g3, srcp, dstp)
    out = _tc_c(p3, g3, h1, h2, dinv_c, emb, b3.reshape(1, D), last_W,
                last_b.reshape(1, D), pred_W, pred_b.reshape(1, OUT), batch_c)
    return out


# speedup vs baseline: 1.9969x; 1.8066x over previous
import functools

import jax
import jax.numpy as jnp
from jax import lax
from jax.experimental import pallas as pl
from jax.experimental.pallas import tpu as pltpu
from jax.experimental.pallas import tpu_sc as plsc

N = 10000
E = 320000
D = 128
G = 128
OUT = 128

NP = 10240
KC = 120
TOTC = 2688
EPAD = TOTC * KC
IR = 4
AC0 = 112
AC1 = 168 - AC0
CH_SC = TOTC // 16
ROWS_T = NP // 16
BM = 1024

_mesh = plsc.VectorSubcoreMesh(core_axis_name="c", subcore_axis_name="s")
_sc_params = pltpu.CompilerParams(needs_layout_passes=False)


def _rsqrt16(x):
    s = 0.5 * (1.0 + x)
    for _ in range(15):
        s = 0.5 * (s + x / s)
    return 1.0 / s


def _zero_fill(buf, nrows):
    def body(i, _):
        for j in range(8):
            buf[i, pl.ds(j * 16, 16)] = jnp.zeros((16,), jnp.float32)
        return 0
    lax.fori_loop(0, nrows, body, 0)


NSEM = 8


def _fire_drain(nchunks, fire):
    full = nchunks // NSEM
    rem = nchunks - full * NSEM

    def grp(g, _):
        base = g * NSEM
        ds_ = [fire(base + k, k) for k in range(NSEM)]
        for dsc in ds_:
            dsc.wait()
        return 0
    lax.fori_loop(0, full, grp, 0)
    ds_ = [fire(full * NSEM + k, k) for k in range(rem)]
    for dsc in ds_:
        dsc.wait()


@functools.partial(
    pl.kernel,
    out_type=(jax.ShapeDtypeStruct((NP,), jnp.float32),
              jax.ShapeDtypeStruct((NP,), jnp.float32)),
    mesh=_mesh,
    compiler_params=_sc_params,
    scratch_types=dict(
        deg_acc=pltpu.VMEM_SHARED((NP,), jnp.float32),
        cs_acc=pltpu.VMEM_SHARED((NP,), jnp.float32),
        dinv_sh=pltpu.VMEM_SHARED((NP,), jnp.float32),
        onesv=pltpu.VMEM((KC,), jnp.float32),
        srcv=pltpu.VMEM((CH_SC, KC), jnp.int32),
        dstv=pltpu.VMEM((CH_SC, KC), jnp.int32),
        valv=pltpu.VMEM((CH_SC, KC), jnp.float32),
        dv=pltpu.VMEM((NP,), jnp.float32),
        dslice=pltpu.VMEM((ROWS_T,), jnp.float32),
        csv=pltpu.VMEM((ROWS_T,), jnp.float32),
        sems=pltpu.SemaphoreType.DMA((NSEM,)),
    ),
)
def _sc_scalar(src_hbm, dst_hbm, dinv_out, cc_out, *, deg_acc, cs_acc,
               dinv_sh, onesv, srcv, dstv, valv, dv, dslice, csv, sems):
    c = lax.axis_index("c")
    s = lax.axis_index("s")

    @pl.when(c == 0)
    def _():
        r0 = s * ROWS_T
        def zb(i, _):
            dslice[pl.ds(i * 16, 16)] = jnp.zeros((16,), jnp.float32)
            return 0
        lax.fori_loop(0, ROWS_T // 16, zb, 0)
        pltpu.sync_copy(dslice, deg_acc.at[pl.ds(r0, ROWS_T)])
        pltpu.sync_copy(dslice, cs_acc.at[pl.ds(r0, ROWS_T)])

        def ob(i, _):
            onesv[pl.ds(i * 16, 16)] = jnp.ones((16,), jnp.float32)
            return 0
        lax.fori_loop(0, KC // 16, ob, 0)
        if KC % 16:
            onesv[pl.ds(KC - 16, 16)] = jnp.ones((16,), jnp.float32)
        plsc.subcore_barrier()

        pltpu.sync_copy(dst_hbm.at[pl.ds(s * CH_SC, CH_SC)], dstv)

        def fire_deg(i, k):
            return pltpu.async_copy(
                onesv, deg_acc.at[dstv.at[i]], sems.at[k], add=True)
        _fire_drain(CH_SC, fire_deg)
        plsc.subcore_barrier()

        pltpu.sync_copy(deg_acc.at[pl.ds(r0, ROWS_T)], csv)

        def rb(i, _):
            x = csv[pl.ds(i * 16, 16)] + 1.0
            dslice[pl.ds(i * 16, 16)] = _rsqrt16(x)
            return 0
        lax.fori_loop(0, ROWS_T // 16, rb, 0)
        pltpu.sync_copy(dslice, dinv_sh.at[pl.ds(r0, ROWS_T)])
        plsc.subcore_barrier()

        pltpu.sync_copy(dinv_sh, dv)
        pltpu.sync_copy(src_hbm.at[pl.ds(s * CH_SC, CH_SC)], srcv)

        def gb(i, _):
            offs = [j * 16 for j in range(KC // 16)]
            if KC % 16:
                offs.append(KC - 16)
            for o in offs:
                idx = srcv[i, pl.ds(o, 16)]
                valv[i, pl.ds(o, 16)] = plsc.load_gather(dv, [idx])
            return 0
        lax.fori_loop(0, CH_SC, gb, 0)

        def fire_cs(i, k):
            return pltpu.async_copy(
                valv.at[i], cs_acc.at[dstv.at[i]], sems.at[k], add=True)
        _fire_drain(CH_SC, fire_cs)
        plsc.subcore_barrier()

        pltpu.sync_copy(cs_acc.at[pl.ds(r0, ROWS_T)], csv)

        def cb(i, _):
            dvv = dslice[pl.ds(i * 16, 16)]
            csv[pl.ds(i * 16, 16)] = dvv * (csv[pl.ds(i * 16, 16)] + dvv)
            return 0
        lax.fori_loop(0, ROWS_T // 16, cb, 0)
        pltpu.sync_copy(dslice, dinv_out.at[pl.ds(r0, ROWS_T)])
        pltpu.sync_copy(csv, cc_out.at[pl.ds(r0, ROWS_T)])


RING = 3
NAGG = 10112
ROWS_A = NAGG // 16


@functools.partial(
    pl.kernel,
    out_type=jax.ShapeDtypeStruct((2, NP, D), jnp.float32),
    mesh=_mesh,
    compiler_params=_sc_params,
    scratch_types=dict(
        acc=pltpu.VMEM_SHARED((NAGG, D), jnp.float32),
        sidx=pltpu.VMEM((IR, KC), jnp.int32),
        didx=pltpu.VMEM((IR, KC), jnp.int32),
        rowbuf=pltpu.VMEM((RING, KC, D), jnp.float32),
        gsems=pltpu.SemaphoreType.DMA((RING,)),
        ssems=pltpu.SemaphoreType.DMA((RING,)),
        isems=pltpu.SemaphoreType.DMA((IR,)),
    ),
)
def _sc_agg(gsc_hbm, src_hbm, dst_hbm, out_hbm, *, acc, sidx, didx, rowbuf,
            gsems, ssems, isems):
    c = lax.axis_index("c")
    s = lax.axis_index("s")
    r0 = s * ROWS_A
    nc = jnp.where(c == 0, AC0, AC1)
    base = jnp.where(c == 0, s * AC0, 16 * AC0 + s * AC1)

    _zero_fill(rowbuf.at[0], KC)
    for k in range(ROWS_A // KC):
        pltpu.sync_copy(rowbuf.at[0], acc.at[pl.ds(r0 + k * KC, KC)])
    rem_rows = ROWS_A - (ROWS_A // KC) * KC
    pltpu.sync_copy(rowbuf.at[0, pl.ds(0, rem_rows)],
                    acc.at[pl.ds(r0 + (ROWS_A // KC) * KC, rem_rows)])
    plsc.subcore_barrier()

    pltpu.sync_copy(src_hbm.at[base], sidx.at[0])
    pltpu.sync_copy(dst_hbm.at[base], didx.at[0])
    pltpu.async_copy(src_hbm.at[base + 1], sidx.at[1], isems.at[1])
    pltpu.async_copy(dst_hbm.at[base + 1], didx.at[1], isems.at[1])
    pltpu.async_copy(gsc_hbm.at[sidx.at[0]], rowbuf.at[0], gsems.at[0])

    def it(i, _):
        b = lax.rem(i, RING)
        nxt = lax.rem(i + 1, RING)
        m1 = lax.rem(i + 1, IR)
        m2 = lax.rem(i + 2, IR)

        @pl.when(jnp.logical_and(i >= 2, i <= nc - 2))
        def _():
            pltpu.make_async_copy(rowbuf.at[nxt], acc.at[didx.at[0]],
                                  ssems.at[nxt]).wait()

        @pl.when(i <= nc - 3)
        def _():
            pltpu.async_copy(src_hbm.at[base + i + 2], sidx.at[m2],
                             isems.at[m2])
            pltpu.async_copy(dst_hbm.at[base + i + 2], didx.at[m2],
                             isems.at[m2])

        @pl.when(i <= nc - 2)
        def _():
            pltpu.make_async_copy(src_hbm.at[base + i + 1], sidx.at[m1],
                                  isems.at[m1]).wait()
            pltpu.make_async_copy(dst_hbm.at[base + i + 1], didx.at[m1],
                                  isems.at[m1]).wait()
            pltpu.async_copy(gsc_hbm.at[sidx.at[m1]], rowbuf.at[nxt],
                             gsems.at[nxt])

        mi = lax.rem(i, IR)
        pltpu.make_async_copy(gsc_hbm.at[sidx.at[mi]], rowbuf.at[b],
                              gsems.at[b]).wait()
        pltpu.async_copy(rowbuf.at[b], acc.at[didx.at[mi]], ssems.at[b],
                         add=True)
        return 0

    lax.fori_loop(0, nc, it, 0)
    for v in range(RING):
        pltpu.make_async_copy(rowbuf.at[v], acc.at[didx.at[0]],
                              ssems.at[v]).wait()

    plsc.subcore_barrier()
    pltpu.sync_copy(acc.at[pl.ds(r0, ROWS_A)],
                    out_hbm.at[c, pl.ds(r0, ROWS_A)])


def _elu(z):
    return jnp.where(z > 0, z, jnp.exp(z) - 1.0)


def _tc_a_body(emb_r, w1_r, b1_r, w2_r, dinv_r, cc_r, h1_r, g2_r):
    e = emb_r[...]
    r1 = jnp.dot(e, w1_r[...], preferred_element_type=jnp.float32)
    h1 = _elu(cc_r[...] * r1 + b1_r[...])
    h1_r[...] = h1
    x2 = h1 + e
    g2_r[...] = dinv_r[...] * jnp.dot(x2, w2_r[...],
                                      preferred_element_type=jnp.float32)


def _tc_a(emb, W1, b1, W2, dinv_c, cc_c):
    grid = (NP // BM,)
    cst = lambda i: (0, 0)
    row = lambda i: (i, 0)
    return pl.pallas_call(
        _tc_a_body,
        grid=grid,
        in_specs=[
            pl.BlockSpec((1, D), cst),
            pl.BlockSpec((D, D), cst),
            pl.BlockSpec((1, D), cst),
            pl.BlockSpec((D, D), cst),
            pl.BlockSpec((BM, 1), row),
            pl.BlockSpec((BM, 1), row),
        ],
        out_specs=[pl.BlockSpec((BM, D), row), pl.BlockSpec((BM, D), row)],
        out_shape=[jax.ShapeDtypeStruct((NP, D), jnp.float32),
                   jax.ShapeDtypeStruct((NP, D), jnp.float32)],
    )(emb, W1, b1, W2, dinv_c, cc_c)


def _tc_b_body(p_r, g2_r, h1_r, dinv_r, emb_r, w3_r, b2_r, h2_r, g3_r):
    agg = dinv_r[...] * (p_r[0] + p_r[1] + g2_r[...])
    h2 = _elu(agg + b2_r[...])
    h2_r[...] = h2
    x3 = emb_r[...] + h1_r[...] + h2
    g3_r[...] = dinv_r[...] * jnp.dot(x3, w3_r[...],
                                      preferred_element_type=jnp.float32)


def _tc_b(p, g2, h1, dinv_c, emb, W3, b2):
    grid = (NP // BM,)
    cst = lambda i: (0, 0)
    row = lambda i: (i, 0)
    return pl.pallas_call(
        _tc_b_body,
        grid=grid,
        in_specs=[
            pl.BlockSpec((2, BM, D), lambda i: (0, i, 0)),
            pl.BlockSpec((BM, D), row),
            pl.BlockSpec((BM, D), row),
            pl.BlockSpec((BM, 1), row),
            pl.BlockSpec((1, D), cst),
            pl.BlockSpec((D, D), cst),
            pl.BlockSpec((1, D), cst),
        ],
        out_specs=[pl.BlockSpec((BM, D), row), pl.BlockSpec((BM, D), row)],
        out_shape=[jax.ShapeDtypeStruct((NP, D), jnp.float32),
                   jax.ShapeDtypeStruct((NP, D), jnp.float32)],
    )(p, g2, h1, dinv_c, emb, W3, b2)


def _tc_c_body(p_r, g3_r, h1_r, h2_r, dinv_r, emb_r, b3_r, lw_r, lb_r, pw_r,
               pb_r, batch_r, out_r, pooled):
    i = pl.program_id(0)
    h3 = _elu(dinv_r[...] * (p_r[0] + p_r[1] + g3_r[...]) + b3_r[...])
    xf = emb_r[...] + h1_r[...] + h2_r[...] + h3
    hf = _elu(jnp.dot(xf, lw_r[...], preferred_element_type=jnp.float32)
              + lb_r[...])
    rid = i * BM + lax.broadcasted_iota(jnp.int32, (BM, 1), 0)
    hf = jnp.where(rid < N, hf, 0.0)
    oh = (batch_r[...] == lax.broadcasted_iota(jnp.int32, (BM, G), 1)
          ).astype(jnp.float32)
    contrib = lax.dot_general(oh, hf, (((0,), (0,)), ((), ())),
                              preferred_element_type=jnp.float32)

    @pl.when(i == 0)
    def _():
        pooled[...] = jnp.zeros((G, D), jnp.float32)

    pooled[...] += contrib
    out_r[...] = (jnp.dot(pooled[...], pw_r[...],
                          preferred_element_type=jnp.float32)
                  + pb_r[...]) * 0.1


def _tc_c(p, g3, h1, h2, dinv_c, emb, b3, last_W, last_b, pred_W, pred_b,
          batch_c):
    grid = (NP // BM,)
    cst = lambda i: (0, 0)
    row = lambda i: (i, 0)
    return pl.pallas_call(
        _tc_c_body,
        grid=grid,
        in_specs=[
            pl.BlockSpec((2, BM, D), lambda i: (0, i, 0)),
            pl.BlockSpec((BM, D), row),
            pl.BlockSpec((BM, D), row),
            pl.BlockSpec((BM, D), row),
            pl.BlockSpec((BM, 1), row),
            pl.BlockSpec((1, D), cst),
            pl.BlockSpec((1, D), cst),
            pl.BlockSpec((D, D), cst),
            pl.BlockSpec((1, D), cst),
            pl.BlockSpec((D, OUT), cst),
            pl.BlockSpec((1, OUT), cst),
            pl.BlockSpec((BM, 1), row),
        ],
        out_specs=pl.BlockSpec((G, OUT), cst),
        out_shape=jax.ShapeDtypeStruct((G, OUT), jnp.float32),
        scratch_shapes=[pltpu.VMEM((G, D), jnp.float32)],
    )(p, g3, h1, h2, dinv_c, emb, b3, last_W, last_b, pred_W, pred_b, batch_c)


def kernel(x, edge_index, batch, edge_attr, emb, W1, b1, W2, b2, W3, b3,
           last_W, last_b, pred_W, pred_b):
    src = edge_index[0]
    dst = edge_index[1]
    srcp = jnp.concatenate(
        [src, jnp.zeros((EPAD - E,), jnp.int32)]).reshape(TOTC, KC)
    dstp = jnp.concatenate(
        [dst, jnp.full((EPAD - E,), N, jnp.int32)]).reshape(TOTC, KC)
    batch_c = jnp.concatenate(
        [batch, jnp.full((NP - N,), G, jnp.int32)]).reshape(NP, 1)

    dinv, cc = _sc_scalar(srcp, dstp)
    dinv_c = dinv.reshape(NP, 1)
    cc_c = cc.reshape(NP, 1)

    h1, g2 = _tc_a(emb, W1, b1.reshape(1, D), W2, dinv_c, cc_c)
    p2 = _sc_agg(g2, srcp, dstp)
    h2, g3 = _tc_b(p2, g2, h1, dinv_c, emb, W3, b2.reshape(1, D))
    p3 = _sc_agg(g3, srcp, dstp)
    out = _tc_c(p3, g3, h1, h2, dinv_c, emb, b3.reshape(1, D), last_W,
                last_b.reshape(1, D), pred_W, pred_b.reshape(1, OUT), batch_c)
    return out


# speedup vs baseline: 2.0678x; 1.0355x over previous
import functools

import jax
import jax.numpy as jnp
from jax import lax
from jax.experimental import pallas as pl
from jax.experimental.pallas import tpu as pltpu
from jax.experimental.pallas import tpu_sc as plsc

N = 10000
E = 320000
D = 128
G = 128
OUT = 128

NP = 10240
KC = 120
TOTC = 2688
EPAD = TOTC * KC
IR = 4
AC0 = 128
AC1 = 168 - AC0
CH_SC = TOTC // 16
ROWS_T = NP // 16
BM = 1024

_mesh = plsc.VectorSubcoreMesh(core_axis_name="c", subcore_axis_name="s")
_sc_params = pltpu.CompilerParams(needs_layout_passes=False)


def _rsqrt16(x):
    s = 0.5 * (1.0 + x)
    for _ in range(15):
        s = 0.5 * (s + x / s)
    return 1.0 / s


def _zero_fill(buf, nrows):
    def body(i, _):
        for j in range(8):
            buf[i, pl.ds(j * 16, 16)] = jnp.zeros((16,), jnp.float32)
        return 0
    lax.fori_loop(0, nrows, body, 0)


NSEM = 8


def _fire_drain(nchunks, fire):
    full = nchunks // NSEM
    rem = nchunks - full * NSEM

    def grp(g, _):
        base = g * NSEM
        ds_ = [fire(base + k, k) for k in range(NSEM)]
        for dsc in ds_:
            dsc.wait()
        return 0
    lax.fori_loop(0, full, grp, 0)
    ds_ = [fire(full * NSEM + k, k) for k in range(rem)]
    for dsc in ds_:
        dsc.wait()


@functools.partial(
    pl.kernel,
    out_type=(jax.ShapeDtypeStruct((NP,), jnp.float32),
              jax.ShapeDtypeStruct((NP,), jnp.float32)),
    mesh=_mesh,
    compiler_params=_sc_params,
    scratch_types=dict(
        deg_acc=pltpu.VMEM_SHARED((NP,), jnp.float32),
        cs_acc=pltpu.VMEM_SHARED((NP,), jnp.float32),
        dinv_sh=pltpu.VMEM_SHARED((NP,), jnp.float32),
        onesv=pltpu.VMEM((KC,), jnp.float32),
        srcv=pltpu.VMEM((CH_SC, KC), jnp.int32),
        dstv=pltpu.VMEM((CH_SC, KC), jnp.int32),
        valv=pltpu.VMEM((CH_SC, KC), jnp.float32),
        dv=pltpu.VMEM((NP,), jnp.float32),
        dslice=pltpu.VMEM((ROWS_T,), jnp.float32),
        csv=pltpu.VMEM((ROWS_T,), jnp.float32),
        sems=pltpu.SemaphoreType.DMA((NSEM,)),
    ),
)
def _sc_scalar(src_hbm, dst_hbm, dinv_out, cc_out, *, deg_acc, cs_acc,
               dinv_sh, onesv, srcv, dstv, valv, dv, dslice, csv, sems):
    c = lax.axis_index("c")
    s = lax.axis_index("s")

    @pl.when(c == 0)
    def _():
        r0 = s * ROWS_T
        def zb(i, _):
            dslice[pl.ds(i * 16, 16)] = jnp.zeros((16,), jnp.float32)
            return 0
        lax.fori_loop(0, ROWS_T // 16, zb, 0)
        pltpu.sync_copy(dslice, deg_acc.at[pl.ds(r0, ROWS_T)])
        pltpu.sync_copy(dslice, cs_acc.at[pl.ds(r0, ROWS_T)])

        def ob(i, _):
            onesv[pl.ds(i * 16, 16)] = jnp.ones((16,), jnp.float32)
            return 0
        lax.fori_loop(0, KC // 16, ob, 0)
        if KC % 16:
            onesv[pl.ds(KC - 16, 16)] = jnp.ones((16,), jnp.float32)
        plsc.subcore_barrier()

        pltpu.sync_copy(dst_hbm.at[pl.ds(s * CH_SC, CH_SC)], dstv)

        def fire_deg(i, k):
            return pltpu.async_copy(
                onesv, deg_acc.at[dstv.at[i]], sems.at[k], add=True)
        _fire_drain(CH_SC, fire_deg)
        plsc.subcore_barrier()

        pltpu.sync_copy(deg_acc.at[pl.ds(r0, ROWS_T)], csv)

        def rb(i, _):
            x = csv[pl.ds(i * 16, 16)] + 1.0
            dslice[pl.ds(i * 16, 16)] = _rsqrt16(x)
            return 0
        lax.fori_loop(0, ROWS_T // 16, rb, 0)
        pltpu.sync_copy(dslice, dinv_sh.at[pl.ds(r0, ROWS_T)])
        plsc.subcore_barrier()

        pltpu.sync_copy(dinv_sh, dv)
        pltpu.sync_copy(src_hbm.at[pl.ds(s * CH_SC, CH_SC)], srcv)

        def gb(i, _):
            offs = [j * 16 for j in range(KC // 16)]
            if KC % 16:
                offs.append(KC - 16)
            for o in offs:
                idx = srcv[i, pl.ds(o, 16)]
                valv[i, pl.ds(o, 16)] = plsc.load_gather(dv, [idx])
            return 0
        lax.fori_loop(0, CH_SC, gb, 0)

        def fire_cs(i, k):
            return pltpu.async_copy(
                valv.at[i], cs_acc.at[dstv.at[i]], sems.at[k], add=True)
        _fire_drain(CH_SC, fire_cs)
        plsc.subcore_barrier()

        pltpu.sync_copy(cs_acc.at[pl.ds(r0, ROWS_T)], csv)

        def cb(i, _):
            dvv = dslice[pl.ds(i * 16, 16)]
            csv[pl.ds(i * 16, 16)] = dvv * (csv[pl.ds(i * 16, 16)] + dvv)
            return 0
        lax.fori_loop(0, ROWS_T // 16, cb, 0)
        pltpu.sync_copy(dslice, dinv_out.at[pl.ds(r0, ROWS_T)])
        pltpu.sync_copy(csv, cc_out.at[pl.ds(r0, ROWS_T)])


RING = 3
NAGG = 10112
ROWS_A = NAGG // 16


@functools.partial(
    pl.kernel,
    out_type=jax.ShapeDtypeStruct((2, NP, D), jnp.float32),
    mesh=_mesh,
    compiler_params=_sc_params,
    scratch_types=dict(
        acc=pltpu.VMEM_SHARED((NAGG, D), jnp.float32),
        sidx=pltpu.VMEM((IR, KC), jnp.int32),
        didx=pltpu.VMEM((IR, KC), jnp.int32),
        rowbuf=pltpu.VMEM((RING, KC, D), jnp.float32),
        gsems=pltpu.SemaphoreType.DMA((RING,)),
        ssems=pltpu.SemaphoreType.DMA((RING,)),
        isems=pltpu.SemaphoreType.DMA((IR,)),
    ),
)
def _sc_agg(gsc_hbm, src_hbm, dst_hbm, out_hbm, *, acc, sidx, didx, rowbuf,
            gsems, ssems, isems):
    c = lax.axis_index("c")
    s = lax.axis_index("s")
    r0 = s * ROWS_A
    nc = jnp.where(c == 0, AC0, AC1)
    base = jnp.where(c == 0, s * AC0, 16 * AC0 + s * AC1)

    _zero_fill(rowbuf.at[0], KC)
    for k in range(ROWS_A // KC):
        pltpu.sync_copy(rowbuf.at[0], acc.at[pl.ds(r0 + k * KC, KC)])
    rem_rows = ROWS_A - (ROWS_A // KC) * KC
    pltpu.sync_copy(rowbuf.at[0, pl.ds(0, rem_rows)],
                    acc.at[pl.ds(r0 + (ROWS_A // KC) * KC, rem_rows)])
    plsc.subcore_barrier()

    pltpu.sync_copy(src_hbm.at[base], sidx.at[0])
    pltpu.sync_copy(dst_hbm.at[base], didx.at[0])
    pltpu.async_copy(src_hbm.at[base + 1], sidx.at[1], isems.at[1])
    pltpu.async_copy(dst_hbm.at[base + 1], didx.at[1], isems.at[1])
    pltpu.async_copy(gsc_hbm.at[sidx.at[0]], rowbuf.at[0], gsems.at[0])

    def it(i, _):
        b = lax.rem(i, RING)
        nxt = lax.rem(i + 1, RING)
        m1 = lax.rem(i + 1, IR)
        m2 = lax.rem(i + 2, IR)

        @pl.when(jnp.logical_and(i >= 2, i <= nc - 2))
        def _():
            pltpu.make_async_copy(rowbuf.at[nxt], acc.at[didx.at[0]],
                                  ssems.at[nxt]).wait()

        @pl.when(i <= nc - 3)
        def _():
            pltpu.async_copy(src_hbm.at[base + i + 2], sidx.at[m2],
                             isems.at[m2])
            pltpu.async_copy(dst_hbm.at[base + i + 2], didx.at[m2],
                             isems.at[m2])

        @pl.when(i <= nc - 2)
        def _():
            pltpu.make_async_copy(src_hbm.at[base + i + 1], sidx.at[m1],
                                  isems.at[m1]).wait()
            pltpu.make_async_copy(dst_hbm.at[base + i + 1], didx.at[m1],
                                  isems.at[m1]).wait()
            pltpu.async_copy(gsc_hbm.at[sidx.at[m1]], rowbuf.at[nxt],
                             gsems.at[nxt])

        mi = lax.rem(i, IR)
        pltpu.make_async_copy(gsc_hbm.at[sidx.at[mi]], rowbuf.at[b],
                              gsems.at[b]).wait()
        pltpu.async_copy(rowbuf.at[b], acc.at[didx.at[mi]], ssems.at[b],
                         add=True)
        return 0

    lax.fori_loop(0, nc, it, 0)
    for v in range(RING):
        pltpu.make_async_copy(rowbuf.at[v], acc.at[didx.at[0]],
                              ssems.at[v]).wait()

    plsc.subcore_barrier()
    pltpu.sync_copy(acc.at[pl.ds(r0, ROWS_A)],
                    out_hbm.at[c, pl.ds(r0, ROWS_A)])


def _elu(z):
    return jnp.where(z > 0, z, jnp.exp(z) - 1.0)


def _tc_a_body(emb_r, w1_r, b1_r, w2_r, dinv_r, cc_r, h1_r, g2_r):
    e = emb_r[...]
    r1 = jnp.dot(e, w1_r[...], preferred_element_type=jnp.float32)
    h1 = _elu(cc_r[...] * r1 + b1_r[...])
    h1_r[...] = h1
    x2 = h1 + e
    g2_r[...] = dinv_r[...] * jnp.dot(x2, w2_r[...],
                                      preferred_element_type=jnp.float32)


def _tc_a(emb, W1, b1, W2, dinv_c, cc_c):
    grid = (NP // BM,)
    cst = lambda i: (0, 0)
    row = lambda i: (i, 0)
    return pl.pallas_call(
        _tc_a_body,
        grid=grid,
        in_specs=[
            pl.BlockSpec((1, D), cst),
            pl.BlockSpec((D, D), cst),
            pl.BlockSpec((1, D), cst),
            pl.BlockSpec((D, D), cst),
            pl.BlockSpec((BM, 1), row),
            pl.BlockSpec((BM, 1), row),
        ],
        out_specs=[pl.BlockSpec((BM, D), row), pl.BlockSpec((BM, D), row)],
        out_shape=[jax.ShapeDtypeStruct((NP, D), jnp.float32),
                   jax.ShapeDtypeStruct((NP, D), jnp.float32)],
    )(emb, W1, b1, W2, dinv_c, cc_c)


def _tc_b_body(p_r, g2_r, h1_r, dinv_r, emb_r, w3_r, b2_r, h2_r, g3_r):
    agg = dinv_r[...] * (p_r[0] + p_r[1] + g2_r[...])
    h2 = _elu(agg + b2_r[...])
    h2_r[...] = h2
    x3 = emb_r[...] + h1_r[...] + h2
    g3_r[...] = dinv_r[...] * jnp.dot(x3, w3_r[...],
                                      preferred_element_type=jnp.float32)


def _tc_b(p, g2, h1, dinv_c, emb, W3, b2):
    grid = (NP // BM,)
    cst = lambda i: (0, 0)
    row = lambda i: (i, 0)
    return pl.pallas_call(
        _tc_b_body,
        grid=grid,
        in_specs=[
            pl.BlockSpec((2, BM, D), lambda i: (0, i, 0)),
            pl.BlockSpec((BM, D), row),
            pl.BlockSpec((BM, D), row),
            pl.BlockSpec((BM, 1), row),
            pl.BlockSpec((1, D), cst),
            pl.BlockSpec((D, D), cst),
            pl.BlockSpec((1, D), cst),
        ],
        out_specs=[pl.BlockSpec((BM, D), row), pl.BlockSpec((BM, D), row)],
        out_shape=[jax.ShapeDtypeStruct((NP, D), jnp.float32),
                   jax.ShapeDtypeStruct((NP, D), jnp.float32)],
    )(p, g2, h1, dinv_c, emb, W3, b2)


def _tc_c_body(p_r, g3_r, h1_r, h2_r, dinv_r, emb_r, b3_r, lw_r, lb_r, pw_r,
               pb_r, batch_r, out_r, pooled):
    i = pl.program_id(0)
    h3 = _elu(dinv_r[...] * (p_r[0] + p_r[1] + g3_r[...]) + b3_r[...])
    xf = emb_r[...] + h1_r[...] + h2_r[...] + h3
    hf = _elu(jnp.dot(xf, lw_r[...], preferred_element_type=jnp.float32)
              + lb_r[...])
    rid = i * BM + lax.broadcasted_iota(jnp.int32, (BM, 1), 0)
    hf = jnp.where(rid < N, hf, 0.0)
    oh = (batch_r[...] == lax.broadcasted_iota(jnp.int32, (BM, G), 1)
          ).astype(jnp.float32)
    contrib = lax.dot_general(oh, hf, (((0,), (0,)), ((), ())),
                              preferred_element_type=jnp.float32)

    @pl.when(i == 0)
    def _():
        pooled[...] = jnp.zeros((G, D), jnp.float32)

    pooled[...] += contrib
    out_r[...] = (jnp.dot(pooled[...], pw_r[...],
                          preferred_element_type=jnp.float32)
                  + pb_r[...]) * 0.1


def _tc_c(p, g3, h1, h2, dinv_c, emb, b3, last_W, last_b, pred_W, pred_b,
          batch_c):
    grid = (NP // BM,)
    cst = lambda i: (0, 0)
    row = lambda i: (i, 0)
    return pl.pallas_call(
        _tc_c_body,
        grid=grid,
        in_specs=[
            pl.BlockSpec((2, BM, D), lambda i: (0, i, 0)),
            pl.BlockSpec((BM, D), row),
            pl.BlockSpec((BM, D), row),
            pl.BlockSpec((BM, D), row),
            pl.BlockSpec((BM, 1), row),
            pl.BlockSpec((1, D), cst),
            pl.BlockSpec((1, D), cst),
            pl.BlockSpec((D, D), cst),
            pl.BlockSpec((1, D), cst),
            pl.BlockSpec((D, OUT), cst),
            pl.BlockSpec((1, OUT), cst),
            pl.BlockSpec((BM, 1), row),
        ],
        out_specs=pl.BlockSpec((G, OUT), cst),
        out_shape=jax.ShapeDtypeStruct((G, OUT), jnp.float32),
        scratch_shapes=[pltpu.VMEM((G, D), jnp.float32)],
    )(p, g3, h1, h2, dinv_c, emb, b3, last_W, last_b, pred_W, pred_b, batch_c)


def kernel(x, edge_index, batch, edge_attr, emb, W1, b1, W2, b2, W3, b3,
           last_W, last_b, pred_W, pred_b):
    src = edge_index[0]
    dst = edge_index[1]
    srcp = jnp.concatenate(
        [src, jnp.zeros((EPAD - E,), jnp.int32)]).reshape(TOTC, KC)
    dstp = jnp.concatenate(
        [dst, jnp.full((EPAD - E,), N, jnp.int32)]).reshape(TOTC, KC)
    batch_c = jnp.concatenate(
        [batch, jnp.full((NP - N,), G, jnp.int32)]).reshape(NP, 1)

    dinv, cc = _sc_scalar(srcp, dstp)
    dinv_c = dinv.reshape(NP, 1)
    cc_c = cc.reshape(NP, 1)

    h1, g2 = _tc_a(emb, W1, b1.reshape(1, D), W2, dinv_c, cc_c)
    p2 = _sc_agg(g2, srcp, dstp)
    h2, g3 = _tc_b(p2, g2, h1, dinv_c, emb, W3, b2.reshape(1, D))
    p3 = _sc_agg(g3, srcp, dstp)
    out = _tc_c(p3, g3, h1, h2, dinv_c, emb, b3.reshape(1, D), last_W,
                last_b.reshape(1, D), pred_W, pred_b.reshape(1, OUT), batch_c)
    return out


# speedup vs baseline: 2.1033x; 1.0172x over previous
import functools

import jax
import jax.numpy as jnp
from jax import lax
from jax.experimental import pallas as pl
from jax.experimental.pallas import tpu as pltpu
from jax.experimental.pallas import tpu_sc as plsc

N = 10000
E = 320000
D = 128
G = 128
OUT = 128

NP = 10240
KC = 120
TOTC = 2688
EPAD = TOTC * KC
IR = 4
AC0 = 136
AC1 = 168 - AC0
CH_SC = TOTC // 16
ROWS_T = NP // 16
BM = 1024

_mesh = plsc.VectorSubcoreMesh(core_axis_name="c", subcore_axis_name="s")
_sc_params = pltpu.CompilerParams(needs_layout_passes=False)


def _rsqrt16(x):
    s = 0.5 * (1.0 + x)
    for _ in range(15):
        s = 0.5 * (s + x / s)
    return 1.0 / s


def _zero_fill(buf, nrows):
    def body(i, _):
        for j in range(8):
            buf[i, pl.ds(j * 16, 16)] = jnp.zeros((16,), jnp.float32)
        return 0
    lax.fori_loop(0, nrows, body, 0)


NSEM = 8


def _fire_drain(nchunks, fire):
    full = nchunks // NSEM
    rem = nchunks - full * NSEM

    def grp(g, _):
        base = g * NSEM
        ds_ = [fire(base + k, k) for k in range(NSEM)]
        for dsc in ds_:
            dsc.wait()
        return 0
    lax.fori_loop(0, full, grp, 0)
    ds_ = [fire(full * NSEM + k, k) for k in range(rem)]
    for dsc in ds_:
        dsc.wait()


@functools.partial(
    pl.kernel,
    out_type=(jax.ShapeDtypeStruct((NP,), jnp.float32),
              jax.ShapeDtypeStruct((NP,), jnp.float32)),
    mesh=_mesh,
    compiler_params=_sc_params,
    scratch_types=dict(
        deg_acc=pltpu.VMEM_SHARED((NP,), jnp.float32),
        cs_acc=pltpu.VMEM_SHARED((NP,), jnp.float32),
        dinv_sh=pltpu.VMEM_SHARED((NP,), jnp.float32),
        onesv=pltpu.VMEM((KC,), jnp.float32),
        srcv=pltpu.VMEM((CH_SC, KC), jnp.int32),
        dstv=pltpu.VMEM((CH_SC, KC), jnp.int32),
        valv=pltpu.VMEM((CH_SC, KC), jnp.float32),
        dv=pltpu.VMEM((NP,), jnp.float32),
        dslice=pltpu.VMEM((ROWS_T,), jnp.float32),
        csv=pltpu.VMEM((ROWS_T,), jnp.float32),
        sems=pltpu.SemaphoreType.DMA((NSEM,)),
    ),
)
def _sc_scalar(src_hbm, dst_hbm, dinv_out, cc_out, *, deg_acc, cs_acc,
               dinv_sh, onesv, srcv, dstv, valv, dv, dslice, csv, sems):
    c = lax.axis_index("c")
    s = lax.axis_index("s")

    @pl.when(c == 0)
    def _():
        r0 = s * ROWS_T
        def zb(i, _):
            dslice[pl.ds(i * 16, 16)] = jnp.zeros((16,), jnp.float32)
            return 0
        lax.fori_loop(0, ROWS_T // 16, zb, 0)
        pltpu.sync_copy(dslice, deg_acc.at[pl.ds(r0, ROWS_T)])
        pltpu.sync_copy(dslice, cs_acc.at[pl.ds(r0, ROWS_T)])

        def ob(i, _):
            onesv[pl.ds(i * 16, 16)] = jnp.ones((16,), jnp.float32)
            return 0
        lax.fori_loop(0, KC // 16, ob, 0)
        if KC % 16:
            onesv[pl.ds(KC - 16, 16)] = jnp.ones((16,), jnp.float32)
        plsc.subcore_barrier()

        pltpu.sync_copy(dst_hbm.at[pl.ds(s * CH_SC, CH_SC)], dstv)

        def fire_deg(i, k):
            return pltpu.async_copy(
                onesv, deg_acc.at[dstv.at[i]], sems.at[k], add=True)
        _fire_drain(CH_SC, fire_deg)
        plsc.subcore_barrier()

        pltpu.sync_copy(deg_acc.at[pl.ds(r0, ROWS_T)], csv)

        def rb(i, _):
            x = csv[pl.ds(i * 16, 16)] + 1.0
            dslice[pl.ds(i * 16, 16)] = _rsqrt16(x)
            return 0
        lax.fori_loop(0, ROWS_T // 16, rb, 0)
        pltpu.sync_copy(dslice, dinv_sh.at[pl.ds(r0, ROWS_T)])
        plsc.subcore_barrier()

        pltpu.sync_copy(dinv_sh, dv)
        pltpu.sync_copy(src_hbm.at[pl.ds(s * CH_SC, CH_SC)], srcv)

        def gb(i, _):
            offs = [j * 16 for j in range(KC // 16)]
            if KC % 16:
                offs.append(KC - 16)
            for o in offs:
                idx = srcv[i, pl.ds(o, 16)]
                valv[i, pl.ds(o, 16)] = plsc.load_gather(dv, [idx])
            return 0
        lax.fori_loop(0, CH_SC, gb, 0)

        def fire_cs(i, k):
            return pltpu.async_copy(
                valv.at[i], cs_acc.at[dstv.at[i]], sems.at[k], add=True)
        _fire_drain(CH_SC, fire_cs)
        plsc.subcore_barrier()

        pltpu.sync_copy(cs_acc.at[pl.ds(r0, ROWS_T)], csv)

        def cb(i, _):
            dvv = dslice[pl.ds(i * 16, 16)]
            csv[pl.ds(i * 16, 16)] = dvv * (csv[pl.ds(i * 16, 16)] + dvv)
            return 0
        lax.fori_loop(0, ROWS_T // 16, cb, 0)
        pltpu.sync_copy(dslice, dinv_out.at[pl.ds(r0, ROWS_T)])
        pltpu.sync_copy(csv, cc_out.at[pl.ds(r0, ROWS_T)])


RING = 3
NAGG = 10112
ROWS_A = NAGG // 16


@functools.partial(
    pl.kernel,
    out_type=jax.ShapeDtypeStruct((2, NP, D), jnp.float32),
    mesh=_mesh,
    compiler_params=_sc_params,
    scratch_types=dict(
        acc=pltpu.VMEM_SHARED((NAGG, D), jnp.float32),
        sidx=pltpu.VMEM((IR, KC), jnp.int32),
        didx=pltpu.VMEM((IR, KC), jnp.int32),
        rowbuf=pltpu.VMEM((RING, KC, D), jnp.float32),
        gsems=pltpu.SemaphoreType.DMA((RING,)),
        ssems=pltpu.SemaphoreType.DMA((RING,)),
        isems=pltpu.SemaphoreType.DMA((IR,)),
    ),
)
def _sc_agg(gsc_hbm, src_hbm, dst_hbm, out_hbm, *, acc, sidx, didx, rowbuf,
            gsems, ssems, isems):
    c = lax.axis_index("c")
    s = lax.axis_index("s")
    r0 = s * ROWS_A
    nc = jnp.where(c == 0, AC0, AC1)
    base = jnp.where(c == 0, s * AC0, 16 * AC0 + s * AC1)

    _zero_fill(rowbuf.at[0], KC)
    for k in range(ROWS_A // KC):
        pltpu.sync_copy(rowbuf.at[0], acc.at[pl.ds(r0 + k * KC, KC)])
    rem_rows = ROWS_A - (ROWS_A // KC) * KC
    pltpu.sync_copy(rowbuf.at[0, pl.ds(0, rem_rows)],
                    acc.at[pl.ds(r0 + (ROWS_A // KC) * KC, rem_rows)])
    plsc.subcore_barrier()

    pltpu.sync_copy(src_hbm.at[base], sidx.at[0])
    pltpu.sync_copy(dst_hbm.at[base], didx.at[0])
    pltpu.async_copy(src_hbm.at[base + 1], sidx.at[1], isems.at[1])
    pltpu.async_copy(dst_hbm.at[base + 1], didx.at[1], isems.at[1])
    pltpu.async_copy(gsc_hbm.at[sidx.at[0]], rowbuf.at[0], gsems.at[0])

    def it(i, _):
        b = lax.rem(i, RING)
        nxt = lax.rem(i + 1, RING)
        m1 = lax.rem(i + 1, IR)
        m2 = lax.rem(i + 2, IR)

        @pl.when(jnp.logical_and(i >= 2, i <= nc - 2))
        def _():
            pltpu.make_async_copy(rowbuf.at[nxt], acc.at[didx.at[0]],
                                  ssems.at[nxt]).wait()

        @pl.when(i <= nc - 3)
        def _():
            pltpu.async_copy(src_hbm.at[base + i + 2], sidx.at[m2],
                             isems.at[m2])
            pltpu.async_copy(dst_hbm.at[base + i + 2], didx.at[m2],
                             isems.at[m2])

        @pl.when(i <= nc - 2)
        def _():
            pltpu.make_async_copy(src_hbm.at[base + i + 1], sidx.at[m1],
                                  isems.at[m1]).wait()
            pltpu.make_async_copy(dst_hbm.at[base + i + 1], didx.at[m1],
                                  isems.at[m1]).wait()
            pltpu.async_copy(gsc_hbm.at[sidx.at[m1]], rowbuf.at[nxt],
                             gsems.at[nxt])

        mi = lax.rem(i, IR)
        pltpu.make_async_copy(gsc_hbm.at[sidx.at[mi]], rowbuf.at[b],
                              gsems.at[b]).wait()
        pltpu.async_copy(rowbuf.at[b], acc.at[didx.at[mi]], ssems.at[b],
                         add=True)
        return 0

    lax.fori_loop(0, nc, it, 0)
    for v in range(RING):
        pltpu.make_async_copy(rowbuf.at[v], acc.at[didx.at[0]],
                              ssems.at[v]).wait()

    plsc.subcore_barrier()
    pltpu.sync_copy(acc.at[pl.ds(r0, ROWS_A)],
                    out_hbm.at[c, pl.ds(r0, ROWS_A)])


def _elu(z):
    return jnp.where(z > 0, z, jnp.exp(z) - 1.0)


def _tc_a_body(emb_r, w1_r, b1_r, w2_r, dinv_r, cc_r, h1_r, g2_r):
    e = emb_r[...]
    r1 = jnp.dot(e, w1_r[...], preferred_element_type=jnp.float32)
    h1 = _elu(cc_r[...] * r1 + b1_r[...])
    h1_r[...] = h1
    x2 = h1 + e
    g2_r[...] = dinv_r[...] * jnp.dot(x2, w2_r[...],
                                      preferred_element_type=jnp.float32)


def _tc_a(emb, W1, b1, W2, dinv_c, cc_c):
    grid = (NP // BM,)
    cst = lambda i: (0, 0)
    row = lambda i: (i, 0)
    return pl.pallas_call(
        _tc_a_body,
        grid=grid,
        in_specs=[
            pl.BlockSpec((1, D), cst),
            pl.BlockSpec((D, D), cst),
            pl.BlockSpec((1, D), cst),
            pl.BlockSpec((D, D), cst),
            pl.BlockSpec((BM, 1), row),
            pl.BlockSpec((BM, 1), row),
        ],
        out_specs=[pl.BlockSpec((BM, D), row), pl.BlockSpec((BM, D), row)],
        out_shape=[jax.ShapeDtypeStruct((NP, D), jnp.float32),
                   jax.ShapeDtypeStruct((NP, D), jnp.float32)],
    )(emb, W1, b1, W2, dinv_c, cc_c)


def _tc_b_body(p_r, g2_r, h1_r, dinv_r, emb_r, w3_r, b2_r, h2_r, g3_r):
    agg = dinv_r[...] * (p_r[0] + p_r[1] + g2_r[...])
    h2 = _elu(agg + b2_r[...])
    h2_r[...] = h2
    x3 = emb_r[...] + h1_r[...] + h2
    g3_r[...] = dinv_r[...] * jnp.dot(x3, w3_r[...],
                                      preferred_element_type=jnp.float32)


def _tc_b(p, g2, h1, dinv_c, emb, W3, b2):
    grid = (NP // BM,)
    cst = lambda i: (0, 0)
    row = lambda i: (i, 0)
    return pl.pallas_call(
        _tc_b_body,
        grid=grid,
        in_specs=[
            pl.BlockSpec((2, BM, D), lambda i: (0, i, 0)),
            pl.BlockSpec((BM, D), row),
            pl.BlockSpec((BM, D), row),
            pl.BlockSpec((BM, 1), row),
            pl.BlockSpec((1, D), cst),
            pl.BlockSpec((D, D), cst),
            pl.BlockSpec((1, D), cst),
        ],
        out_specs=[pl.BlockSpec((BM, D), row), pl.BlockSpec((BM, D), row)],
        out_shape=[jax.ShapeDtypeStruct((NP, D), jnp.float32),
                   jax.ShapeDtypeStruct((NP, D), jnp.float32)],
    )(p, g2, h1, dinv_c, emb, W3, b2)


def _tc_c_body(p_r, g3_r, h1_r, h2_r, dinv_r, emb_r, b3_r, lw_r, lb_r, pw_r,
               pb_r, batch_r, out_r, pooled):
    i = pl.program_id(0)
    h3 = _elu(dinv_r[...] * (p_r[0] + p_r[1] + g3_r[...]) + b3_r[...])
    xf = emb_r[...] + h1_r[...] + h2_r[...] + h3
    hf = _elu(jnp.dot(xf, lw_r[...], preferred_element_type=jnp.float32)
              + lb_r[...])
    rid = i * BM + lax.broadcasted_iota(jnp.int32, (BM, 1), 0)
    hf = jnp.where(rid < N, hf, 0.0)
    oh = (batch_r[...] == lax.broadcasted_iota(jnp.int32, (BM, G), 1)
          ).astype(jnp.float32)
    contrib = lax.dot_general(oh, hf, (((0,), (0,)), ((), ())),
                              preferred_element_type=jnp.float32)

    @pl.when(i == 0)
    def _():
        pooled[...] = jnp.zeros((G, D), jnp.float32)

    pooled[...] += contrib
    out_r[...] = (jnp.dot(pooled[...], pw_r[...],
                          preferred_element_type=jnp.float32)
                  + pb_r[...]) * 0.1


def _tc_c(p, g3, h1, h2, dinv_c, emb, b3, last_W, last_b, pred_W, pred_b,
          batch_c):
    grid = (NP // BM,)
    cst = lambda i: (0, 0)
    row = lambda i: (i, 0)
    return pl.pallas_call(
        _tc_c_body,
        grid=grid,
        in_specs=[
            pl.BlockSpec((2, BM, D), lambda i: (0, i, 0)),
            pl.BlockSpec((BM, D), row),
            pl.BlockSpec((BM, D), row),
            pl.BlockSpec((BM, D), row),
            pl.BlockSpec((BM, 1), row),
            pl.BlockSpec((1, D), cst),
            pl.BlockSpec((1, D), cst),
            pl.BlockSpec((D, D), cst),
            pl.BlockSpec((1, D), cst),
            pl.BlockSpec((D, OUT), cst),
            pl.BlockSpec((1, OUT), cst),
            pl.BlockSpec((BM, 1), row),
        ],
        out_specs=pl.BlockSpec((G, OUT), cst),
        out_shape=jax.ShapeDtypeStruct((G, OUT), jnp.float32),
        scratch_shapes=[pltpu.VMEM((G, D), jnp.float32)],
    )(p, g3, h1, h2, dinv_c, emb, b3, last_W, last_b, pred_W, pred_b, batch_c)


def kernel(x, edge_index, batch, edge_attr, emb, W1, b1, W2, b2, W3, b3,
           last_W, last_b, pred_W, pred_b):
    src = edge_index[0]
    dst = edge_index[1]
    srcp = jnp.concatenate(
        [src, jnp.zeros((EPAD - E,), jnp.int32)]).reshape(TOTC, KC)
    dstp = jnp.concatenate(
        [dst, jnp.full((EPAD - E,), N, jnp.int32)]).reshape(TOTC, KC)
    batch_c = jnp.concatenate(
        [batch, jnp.full((NP - N,), G, jnp.int32)]).reshape(NP, 1)

    dinv, cc = _sc_scalar(srcp, dstp)
    dinv_c = dinv.reshape(NP, 1)
    cc_c = cc.reshape(NP, 1)

    h1, g2 = _tc_a(emb, W1, b1.reshape(1, D), W2, dinv_c, cc_c)
    p2 = _sc_agg(g2, srcp, dstp)
    h2, g3 = _tc_b(p2, g2, h1, dinv_c, emb, W3, b2.reshape(1, D))
    p3 = _sc_agg(g3, srcp, dstp)
    out = _tc_c(p3, g3, h1, h2, dinv_c, emb, b3.reshape(1, D), last_W,
                last_b.reshape(1, D), pred_W, pred_b.reshape(1, OUT), batch_c)
    return out


# speedup vs baseline: 2.1034x; 1.0001x over previous
import functools

import jax
import jax.numpy as jnp
from jax import lax
from jax.experimental import pallas as pl
from jax.experimental.pallas import tpu as pltpu
from jax.experimental.pallas import tpu_sc as plsc

N = 10000
E = 320000
D = 128
G = 128
OUT = 128

NP = 10240
KC = 120
TOTC = 2688
EPAD = TOTC * KC
IR = 5
AC0 = 136
AC1 = 168 - AC0
CH_SC = TOTC // 16
ROWS_T = NP // 16
BM = 1024

_mesh = plsc.VectorSubcoreMesh(core_axis_name="c", subcore_axis_name="s")
_sc_params = pltpu.CompilerParams(needs_layout_passes=False)


def _rsqrt16(x):
    s = 0.5 * (1.0 + x)
    for _ in range(15):
        s = 0.5 * (s + x / s)
    return 1.0 / s


def _zero_fill(buf, nrows):
    def body(i, _):
        for j in range(8):
            buf[i, pl.ds(j * 16, 16)] = jnp.zeros((16,), jnp.float32)
        return 0
    lax.fori_loop(0, nrows, body, 0)


NSEM = 8


def _fire_drain(nchunks, fire):
    full = nchunks // NSEM
    rem = nchunks - full * NSEM

    def grp(g, _):
        base = g * NSEM
        ds_ = [fire(base + k, k) for k in range(NSEM)]
        for dsc in ds_:
            dsc.wait()
        return 0
    lax.fori_loop(0, full, grp, 0)
    ds_ = [fire(full * NSEM + k, k) for k in range(rem)]
    for dsc in ds_:
        dsc.wait()


@functools.partial(
    pl.kernel,
    out_type=(jax.ShapeDtypeStruct((NP,), jnp.float32),
              jax.ShapeDtypeStruct((NP,), jnp.float32)),
    mesh=_mesh,
    compiler_params=_sc_params,
    scratch_types=dict(
        deg_acc=pltpu.VMEM_SHARED((NP,), jnp.float32),
        cs_acc=pltpu.VMEM_SHARED((NP,), jnp.float32),
        dinv_sh=pltpu.VMEM_SHARED((NP,), jnp.float32),
        onesv=pltpu.VMEM((KC,), jnp.float32),
        srcv=pltpu.VMEM((CH_SC, KC), jnp.int32),
        dstv=pltpu.VMEM((CH_SC, KC), jnp.int32),
        valv=pltpu.VMEM((CH_SC, KC), jnp.float32),
        dv=pltpu.VMEM((NP,), jnp.float32),
        dslice=pltpu.VMEM((ROWS_T,), jnp.float32),
        csv=pltpu.VMEM((ROWS_T,), jnp.float32),
        sems=pltpu.SemaphoreType.DMA((NSEM,)),
    ),
)
def _sc_scalar(src_hbm, dst_hbm, dinv_out, cc_out, *, deg_acc, cs_acc,
               dinv_sh, onesv, srcv, dstv, valv, dv, dslice, csv, sems):
    c = lax.axis_index("c")
    s = lax.axis_index("s")

    @pl.when(c == 0)
    def _():
        r0 = s * ROWS_T
        def zb(i, _):
            dslice[pl.ds(i * 16, 16)] = jnp.zeros((16,), jnp.float32)
            return 0
        lax.fori_loop(0, ROWS_T // 16, zb, 0)
        pltpu.sync_copy(dslice, deg_acc.at[pl.ds(r0, ROWS_T)])
        pltpu.sync_copy(dslice, cs_acc.at[pl.ds(r0, ROWS_T)])

        def ob(i, _):
            onesv[pl.ds(i * 16, 16)] = jnp.ones((16,), jnp.float32)
            return 0
        lax.fori_loop(0, KC // 16, ob, 0)
        if KC % 16:
            onesv[pl.ds(KC - 16, 16)] = jnp.ones((16,), jnp.float32)
        plsc.subcore_barrier()

        pltpu.sync_copy(dst_hbm.at[pl.ds(s * CH_SC, CH_SC)], dstv)

        def fire_deg(i, k):
            return pltpu.async_copy(
                onesv, deg_acc.at[dstv.at[i]], sems.at[k], add=True)
        _fire_drain(CH_SC, fire_deg)
        plsc.subcore_barrier()

        pltpu.sync_copy(deg_acc.at[pl.ds(r0, ROWS_T)], csv)

        def rb(i, _):
            x = csv[pl.ds(i * 16, 16)] + 1.0
            dslice[pl.ds(i * 16, 16)] = _rsqrt16(x)
            return 0
        lax.fori_loop(0, ROWS_T // 16, rb, 0)
        pltpu.sync_copy(dslice, dinv_sh.at[pl.ds(r0, ROWS_T)])
        plsc.subcore_barrier()

        pltpu.sync_copy(dinv_sh, dv)
        pltpu.sync_copy(src_hbm.at[pl.ds(s * CH_SC, CH_SC)], srcv)

        def gb(i, _):
            offs = [j * 16 for j in range(KC // 16)]
            if KC % 16:
                offs.append(KC - 16)
            for o in offs:
                idx = srcv[i, pl.ds(o, 16)]
                valv[i, pl.ds(o, 16)] = plsc.load_gather(dv, [idx])
            return 0
        lax.fori_loop(0, CH_SC, gb, 0)

        def fire_cs(i, k):
            return pltpu.async_copy(
                valv.at[i], cs_acc.at[dstv.at[i]], sems.at[k], add=True)
        _fire_drain(CH_SC, fire_cs)
        plsc.subcore_barrier()

        pltpu.sync_copy(cs_acc.at[pl.ds(r0, ROWS_T)], csv)

        def cb(i, _):
            dvv = dslice[pl.ds(i * 16, 16)]
            csv[pl.ds(i * 16, 16)] = dvv * (csv[pl.ds(i * 16, 16)] + dvv)
            return 0
        lax.fori_loop(0, ROWS_T // 16, cb, 0)
        pltpu.sync_copy(dslice, dinv_out.at[pl.ds(r0, ROWS_T)])
        pltpu.sync_copy(csv, cc_out.at[pl.ds(r0, ROWS_T)])


RING = 3
NAGG = 10112
ROWS_A = NAGG // 16


@functools.partial(
    pl.kernel,
    out_type=jax.ShapeDtypeStruct((2, NP, D), jnp.float32),
    mesh=_mesh,
    compiler_params=_sc_params,
    scratch_types=dict(
        acc=pltpu.VMEM_SHARED((NAGG, D), jnp.float32),
        sidx=pltpu.VMEM((IR, KC), jnp.int32),
        didx=pltpu.VMEM((IR, KC), jnp.int32),
        rowbuf=pltpu.VMEM((RING, KC, D), jnp.float32),
        gsems=pltpu.SemaphoreType.DMA((RING,)),
        ssems=pltpu.SemaphoreType.DMA((RING,)),
        isems=pltpu.SemaphoreType.DMA((IR,)),
    ),
)
def _sc_agg(gsc_hbm, src_hbm, dst_hbm, out_hbm, *, acc, sidx, didx, rowbuf,
            gsems, ssems, isems):
    c = lax.axis_index("c")
    s = lax.axis_index("s")
    r0 = s * ROWS_A
    nc = jnp.where(c == 0, AC0, AC1)
    base = jnp.where(c == 0, s * AC0, 16 * AC0 + s * AC1)

    _zero_fill(rowbuf.at[0], KC)
    for k in range(ROWS_A // KC):
        pltpu.sync_copy(rowbuf.at[0], acc.at[pl.ds(r0 + k * KC, KC)])
    rem_rows = ROWS_A - (ROWS_A // KC) * KC
    pltpu.sync_copy(rowbuf.at[0, pl.ds(0, rem_rows)],
                    acc.at[pl.ds(r0 + (ROWS_A // KC) * KC, rem_rows)])
    plsc.subcore_barrier()

    pltpu.sync_copy(src_hbm.at[base], sidx.at[0])
    pltpu.sync_copy(dst_hbm.at[base], didx.at[0])
    for k in (1, 2):
        pltpu.async_copy(src_hbm.at[base + k], sidx.at[k], isems.at[k])
        pltpu.async_copy(dst_hbm.at[base + k], didx.at[k], isems.at[k])
    pltpu.async_copy(gsc_hbm.at[sidx.at[0]], rowbuf.at[0], gsems.at[0])

    def it(i, _):
        b = lax.rem(i, RING)
        nxt = lax.rem(i + 1, RING)
        m1 = lax.rem(i + 1, IR)
        m3 = lax.rem(i + 3, IR)

        @pl.when(jnp.logical_and(i >= 2, i <= nc - 2))
        def _():
            pltpu.make_async_copy(rowbuf.at[nxt], acc.at[didx.at[0]],
                                  ssems.at[nxt]).wait()

        @pl.when(i <= nc - 4)
        def _():
            pltpu.async_copy(src_hbm.at[base + i + 3], sidx.at[m3],
                             isems.at[m3])
            pltpu.async_copy(dst_hbm.at[base + i + 3], didx.at[m3],
                             isems.at[m3])

        @pl.when(i <= nc - 2)
        def _():
            pltpu.make_async_copy(src_hbm.at[base + i + 1], sidx.at[m1],
                                  isems.at[m1]).wait()
            pltpu.make_async_copy(dst_hbm.at[base + i + 1], didx.at[m1],
                                  isems.at[m1]).wait()
            pltpu.async_copy(gsc_hbm.at[sidx.at[m1]], rowbuf.at[nxt],
                             gsems.at[nxt])

        mi = lax.rem(i, IR)
        pltpu.make_async_copy(gsc_hbm.at[sidx.at[mi]], rowbuf.at[b],
                              gsems.at[b]).wait()
        pltpu.async_copy(rowbuf.at[b], acc.at[didx.at[mi]], ssems.at[b],
                         add=True)
        return 0

    lax.fori_loop(0, nc, it, 0)
    for v in range(RING):
        pltpu.make_async_copy(rowbuf.at[v], acc.at[didx.at[0]],
                              ssems.at[v]).wait()

    plsc.subcore_barrier()
    pltpu.sync_copy(acc.at[pl.ds(r0, ROWS_A)],
                    out_hbm.at[c, pl.ds(r0, ROWS_A)])


def _elu(z):
    return jnp.where(z > 0, z, jnp.exp(z) - 1.0)


def _tc_a_body(emb_r, w1_r, b1_r, w2_r, dinv_r, cc_r, h1_r, g2_r):
    e = emb_r[...]
    r1 = jnp.dot(e, w1_r[...], preferred_element_type=jnp.float32)
    h1 = _elu(cc_r[...] * r1 + b1_r[...])
    h1_r[...] = h1
    x2 = h1 + e
    g2_r[...] = dinv_r[...] * jnp.dot(x2, w2_r[...],
                                      preferred_element_type=jnp.float32)


def _tc_a(emb, W1, b1, W2, dinv_c, cc_c):
    grid = (NP // BM,)
    cst = lambda i: (0, 0)
    row = lambda i: (i, 0)
    return pl.pallas_call(
        _tc_a_body,
        grid=grid,
        in_specs=[
            pl.BlockSpec((1, D), cst),
            pl.BlockSpec((D, D), cst),
            pl.BlockSpec((1, D), cst),
            pl.BlockSpec((D, D), cst),
            pl.BlockSpec((BM, 1), row),
            pl.BlockSpec((BM, 1), row),
        ],
        out_specs=[pl.BlockSpec((BM, D), row), pl.BlockSpec((BM, D), row)],
        out_shape=[jax.ShapeDtypeStruct((NP, D), jnp.float32),
                   jax.ShapeDtypeStruct((NP, D), jnp.float32)],
    )(emb, W1, b1, W2, dinv_c, cc_c)


def _tc_b_body(p_r, g2_r, h1_r, dinv_r, emb_r, w3_r, b2_r, h2_r, g3_r):
    agg = dinv_r[...] * (p_r[0] + p_r[1] + g2_r[...])
    h2 = _elu(agg + b2_r[...])
    h2_r[...] = h2
    x3 = emb_r[...] + h1_r[...] + h2
    g3_r[...] = dinv_r[...] * jnp.dot(x3, w3_r[...],
                                      preferred_element_type=jnp.float32)


def _tc_b(p, g2, h1, dinv_c, emb, W3, b2):
    grid = (NP // BM,)
    cst = lambda i: (0, 0)
    row = lambda i: (i, 0)
    return pl.pallas_call(
        _tc_b_body,
        grid=grid,
        in_specs=[
            pl.BlockSpec((2, BM, D), lambda i: (0, i, 0)),
            pl.BlockSpec((BM, D), row),
            pl.BlockSpec((BM, D), row),
            pl.BlockSpec((BM, 1), row),
            pl.BlockSpec((1, D), cst),
            pl.BlockSpec((D, D), cst),
            pl.BlockSpec((1, D), cst),
        ],
        out_specs=[pl.BlockSpec((BM, D), row), pl.BlockSpec((BM, D), row)],
        out_shape=[jax.ShapeDtypeStruct((NP, D), jnp.float32),
                   jax.ShapeDtypeStruct((NP, D), jnp.float32)],
    )(p, g2, h1, dinv_c, emb, W3, b2)


def _tc_c_body(p_r, g3_r, h1_r, h2_r, dinv_r, emb_r, b3_r, lw_r, lb_r, pw_r,
               pb_r, batch_r, out_r, pooled):
    i = pl.program_id(0)
    h3 = _elu(dinv_r[...] * (p_r[0] + p_r[1] + g3_r[...]) + b3_r[...])
    xf = emb_r[...] + h1_r[...] + h2_r[...] + h3
    hf = _elu(jnp.dot(xf, lw_r[...], preferred_element_type=jnp.float32)
              + lb_r[...])
    rid = i * BM + lax.broadcasted_iota(jnp.int32, (BM, 1), 0)
    hf = jnp.where(rid < N, hf, 0.0)
    oh = (batch_r[...] == lax.broadcasted_iota(jnp.int32, (BM, G), 1)
          ).astype(jnp.float32)
    contrib = lax.dot_general(oh, hf, (((0,), (0,)), ((), ())),
                              preferred_element_type=jnp.float32)

    @pl.when(i == 0)
    def _():
        pooled[...] = jnp.zeros((G, D), jnp.float32)

    pooled[...] += contrib
    out_r[...] = (jnp.dot(pooled[...], pw_r[...],
                          preferred_element_type=jnp.float32)
                  + pb_r[...]) * 0.1


def _tc_c(p, g3, h1, h2, dinv_c, emb, b3, last_W, last_b, pred_W, pred_b,
          batch_c):
    grid = (NP // BM,)
    cst = lambda i: (0, 0)
    row = lambda i: (i, 0)
    return pl.pallas_call(
        _tc_c_body,
        grid=grid,
        in_specs=[
            pl.BlockSpec((2, BM, D), lambda i: (0, i, 0)),
            pl.BlockSpec((BM, D), row),
            pl.BlockSpec((BM, D), row),
            pl.BlockSpec((BM, D), row),
            pl.BlockSpec((BM, 1), row),
            pl.BlockSpec((1, D), cst),
            pl.BlockSpec((1, D), cst),
            pl.BlockSpec((D, D), cst),
            pl.BlockSpec((1, D), cst),
            pl.BlockSpec((D, OUT), cst),
            pl.BlockSpec((1, OUT), cst),
            pl.BlockSpec((BM, 1), row),
        ],
        out_specs=pl.BlockSpec((G, OUT), cst),
        out_shape=jax.ShapeDtypeStruct((G, OUT), jnp.float32),
        scratch_shapes=[pltpu.VMEM((G, D), jnp.float32)],
    )(p, g3, h1, h2, dinv_c, emb, b3, last_W, last_b, pred_W, pred_b, batch_c)


def kernel(x, edge_index, batch, edge_attr, emb, W1, b1, W2, b2, W3, b3,
           last_W, last_b, pred_W, pred_b):
    src = edge_index[0]
    dst = edge_index[1]
    srcp = jnp.concatenate(
        [src, jnp.zeros((EPAD - E,), jnp.int32)]).reshape(TOTC, KC)
    dstp = jnp.concatenate(
        [dst, jnp.full((EPAD - E,), N, jnp.int32)]).reshape(TOTC, KC)
    batch_c = jnp.concatenate(
        [batch, jnp.full((NP - N,), G, jnp.int32)]).reshape(NP, 1)

    dinv, cc = _sc_scalar(srcp, dstp)
    dinv_c = dinv.reshape(NP, 1)
    cc_c = cc.reshape(NP, 1)

    h1, g2 = _tc_a(emb, W1, b1.reshape(1, D), W2, dinv_c, cc_c)
    p2 = _sc_agg(g2, srcp, dstp)
    h2, g3 = _tc_b(p2, g2, h1, dinv_c, emb, W3, b2.reshape(1, D))
    p3 = _sc_agg(g3, srcp, dstp)
    out = _tc_c(p3, g3, h1, h2, dinv_c, emb, b3.reshape(1, D), last_W,
                last_b.reshape(1, D), pred_W, pred_b.reshape(1, OUT), batch_c)
    return out


# speedup vs baseline: 2.1296x; 1.0125x over previous
import functools

import jax
import jax.numpy as jnp
from jax import lax
from jax.experimental import pallas as pl
from jax.experimental.pallas import tpu as pltpu
from jax.experimental.pallas import tpu_sc as plsc

N = 10000
E = 320000
D = 128
G = 128
OUT = 128

NP = 10240
KC = 120
TOTC = 2688
EPAD = TOTC * KC
IR = 5
AC0 = 136
AC1 = 168 - AC0
CH_SC = TOTC // 16
ROWS_T = NP // 16
BM = 2048

_mesh = plsc.VectorSubcoreMesh(core_axis_name="c", subcore_axis_name="s")
_sc_params = pltpu.CompilerParams(needs_layout_passes=False)


def _rsqrt16(x):
    s = 0.5 * (1.0 + x)
    for _ in range(15):
        s = 0.5 * (s + x / s)
    return 1.0 / s


def _zero_fill(buf, nrows):
    def body(i, _):
        for j in range(8):
            buf[i, pl.ds(j * 16, 16)] = jnp.zeros((16,), jnp.float32)
        return 0
    lax.fori_loop(0, nrows, body, 0)


NSEM = 8


def _fire_drain(nchunks, fire):
    full = nchunks // NSEM
    rem = nchunks - full * NSEM

    def grp(g, _):
        base = g * NSEM
        ds_ = [fire(base + k, k) for k in range(NSEM)]
        for dsc in ds_:
            dsc.wait()
        return 0
    lax.fori_loop(0, full, grp, 0)
    ds_ = [fire(full * NSEM + k, k) for k in range(rem)]
    for dsc in ds_:
        dsc.wait()


@functools.partial(
    pl.kernel,
    out_type=(jax.ShapeDtypeStruct((NP,), jnp.float32),
              jax.ShapeDtypeStruct((NP,), jnp.float32)),
    mesh=_mesh,
    compiler_params=_sc_params,
    scratch_types=dict(
        deg_acc=pltpu.VMEM_SHARED((NP,), jnp.float32),
        cs_acc=pltpu.VMEM_SHARED((NP,), jnp.float32),
        dinv_sh=pltpu.VMEM_SHARED((NP,), jnp.float32),
        onesv=pltpu.VMEM((KC,), jnp.float32),
        srcv=pltpu.VMEM((CH_SC, KC), jnp.int32),
        dstv=pltpu.VMEM((CH_SC, KC), jnp.int32),
        valv=pltpu.VMEM((CH_SC, KC), jnp.float32),
        dv=pltpu.VMEM((NP,), jnp.float32),
        dslice=pltpu.VMEM((ROWS_T,), jnp.float32),
        csv=pltpu.VMEM((ROWS_T,), jnp.float32),
        sems=pltpu.SemaphoreType.DMA((NSEM,)),
    ),
)
def _sc_scalar(src_hbm, dst_hbm, dinv_out, cc_out, *, deg_acc, cs_acc,
               dinv_sh, onesv, srcv, dstv, valv, dv, dslice, csv, sems):
    c = lax.axis_index("c")
    s = lax.axis_index("s")

    @pl.when(c == 0)
    def _():
        r0 = s * ROWS_T
        def zb(i, _):
            dslice[pl.ds(i * 16, 16)] = jnp.zeros((16,), jnp.float32)
            return 0
        lax.fori_loop(0, ROWS_T // 16, zb, 0)
        pltpu.sync_copy(dslice, deg_acc.at[pl.ds(r0, ROWS_T)])
        pltpu.sync_copy(dslice, cs_acc.at[pl.ds(r0, ROWS_T)])

        def ob(i, _):
            onesv[pl.ds(i * 16, 16)] = jnp.ones((16,), jnp.float32)
            return 0
        lax.fori_loop(0, KC // 16, ob, 0)
        if KC % 16:
            onesv[pl.ds(KC - 16, 16)] = jnp.ones((16,), jnp.float32)
        plsc.subcore_barrier()

        pltpu.sync_copy(dst_hbm.at[pl.ds(s * CH_SC, CH_SC)], dstv)

        def fire_deg(i, k):
            return pltpu.async_copy(
                onesv, deg_acc.at[dstv.at[i]], sems.at[k], add=True)
        _fire_drain(CH_SC, fire_deg)
        plsc.subcore_barrier()

        pltpu.sync_copy(deg_acc.at[pl.ds(r0, ROWS_T)], csv)

        def rb(i, _):
            x = csv[pl.ds(i * 16, 16)] + 1.0
            dslice[pl.ds(i * 16, 16)] = _rsqrt16(x)
            return 0
        lax.fori_loop(0, ROWS_T // 16, rb, 0)
        pltpu.sync_copy(dslice, dinv_sh.at[pl.ds(r0, ROWS_T)])
        plsc.subcore_barrier()

        pltpu.sync_copy(dinv_sh, dv)
        pltpu.sync_copy(src_hbm.at[pl.ds(s * CH_SC, CH_SC)], srcv)

        def gb(i, _):
            offs = [j * 16 for j in range(KC // 16)]
            if KC % 16:
                offs.append(KC - 16)
            for o in offs:
                idx = srcv[i, pl.ds(o, 16)]
                valv[i, pl.ds(o, 16)] = plsc.load_gather(dv, [idx])
            return 0
        lax.fori_loop(0, CH_SC, gb, 0)

        def fire_cs(i, k):
            return pltpu.async_copy(
                valv.at[i], cs_acc.at[dstv.at[i]], sems.at[k], add=True)
        _fire_drain(CH_SC, fire_cs)
        plsc.subcore_barrier()

        pltpu.sync_copy(cs_acc.at[pl.ds(r0, ROWS_T)], csv)

        def cb(i, _):
            dvv = dslice[pl.ds(i * 16, 16)]
            csv[pl.ds(i * 16, 16)] = dvv * (csv[pl.ds(i * 16, 16)] + dvv)
            return 0
        lax.fori_loop(0, ROWS_T // 16, cb, 0)
        pltpu.sync_copy(dslice, dinv_out.at[pl.ds(r0, ROWS_T)])
        pltpu.sync_copy(csv, cc_out.at[pl.ds(r0, ROWS_T)])


RING = 3
NAGG = 10112
ROWS_A = NAGG // 16


@functools.partial(
    pl.kernel,
    out_type=jax.ShapeDtypeStruct((2, NP, D), jnp.float32),
    mesh=_mesh,
    compiler_params=_sc_params,
    scratch_types=dict(
        acc=pltpu.VMEM_SHARED((NAGG, D), jnp.float32),
        sidx=pltpu.VMEM((IR, KC), jnp.int32),
        didx=pltpu.VMEM((IR, KC), jnp.int32),
        rowbuf=pltpu.VMEM((RING, KC, D), jnp.float32),
        gsems=pltpu.SemaphoreType.DMA((RING,)),
        ssems=pltpu.SemaphoreType.DMA((RING,)),
        isems=pltpu.SemaphoreType.DMA((IR,)),
    ),
)
def _sc_agg(gsc_hbm, src_hbm, dst_hbm, out_hbm, *, acc, sidx, didx, rowbuf,
            gsems, ssems, isems):
    c = lax.axis_index("c")
    s = lax.axis_index("s")
    r0 = s * ROWS_A
    nc = jnp.where(c == 0, AC0, AC1)
    base = jnp.where(c == 0, s * AC0, 16 * AC0 + s * AC1)

    _zero_fill(rowbuf.at[0], KC)
    for k in range(ROWS_A // KC):
        pltpu.sync_copy(rowbuf.at[0], acc.at[pl.ds(r0 + k * KC, KC)])
    rem_rows = ROWS_A - (ROWS_A // KC) * KC
    pltpu.sync_copy(rowbuf.at[0, pl.ds(0, rem_rows)],
                    acc.at[pl.ds(r0 + (ROWS_A // KC) * KC, rem_rows)])
    plsc.subcore_barrier()

    pltpu.sync_copy(src_hbm.at[base], sidx.at[0])
    pltpu.sync_copy(dst_hbm.at[base], didx.at[0])
    for k in (1, 2):
        pltpu.async_copy(src_hbm.at[base + k], sidx.at[k], isems.at[k])
        pltpu.async_copy(dst_hbm.at[base + k], didx.at[k], isems.at[k])
    pltpu.async_copy(gsc_hbm.at[sidx.at[0]], rowbuf.at[0], gsems.at[0])

    def it(i, _):
        b = lax.rem(i, RING)
        nxt = lax.rem(i + 1, RING)
        m1 = lax.rem(i + 1, IR)
        m3 = lax.rem(i + 3, IR)

        @pl.when(jnp.logical_and(i >= 2, i <= nc - 2))
        def _():
            pltpu.make_async_copy(rowbuf.at[nxt], acc.at[didx.at[0]],
                                  ssems.at[nxt]).wait()

        @pl.when(i <= nc - 4)
        def _():
            pltpu.async_copy(src_hbm.at[base + i + 3], sidx.at[m3],
                             isems.at[m3])
            pltpu.async_copy(dst_hbm.at[base + i + 3], didx.at[m3],
                             isems.at[m3])

        @pl.when(i <= nc - 2)
        def _():
            pltpu.make_async_copy(src_hbm.at[base + i + 1], sidx.at[m1],
                                  isems.at[m1]).wait()
            pltpu.make_async_copy(dst_hbm.at[base + i + 1], didx.at[m1],
                                  isems.at[m1]).wait()
            pltpu.async_copy(gsc_hbm.at[sidx.at[m1]], rowbuf.at[nxt],
                             gsems.at[nxt])

        mi = lax.rem(i, IR)
        pltpu.make_async_copy(gsc_hbm.at[sidx.at[mi]], rowbuf.at[b],
                              gsems.at[b]).wait()
        pltpu.async_copy(rowbuf.at[b], acc.at[didx.at[mi]], ssems.at[b],
                         add=True)
        return 0

    lax.fori_loop(0, nc, it, 0)
    for v in range(RING):
        pltpu.make_async_copy(rowbuf.at[v], acc.at[didx.at[0]],
                              ssems.at[v]).wait()

    plsc.subcore_barrier()
    pltpu.sync_copy(acc.at[pl.ds(r0, ROWS_A)],
                    out_hbm.at[c, pl.ds(r0, ROWS_A)])


def _elu(z):
    return jnp.where(z > 0, z, jnp.exp(z) - 1.0)


def _tc_a_body(emb_r, w1_r, b1_r, w2_r, dinv_r, cc_r, h1_r, g2_r):
    e = emb_r[...]
    r1 = jnp.dot(e, w1_r[...], preferred_element_type=jnp.float32)
    h1 = _elu(cc_r[...] * r1 + b1_r[...])
    h1_r[...] = h1
    x2 = h1 + e
    g2_r[...] = dinv_r[...] * jnp.dot(x2, w2_r[...],
                                      preferred_element_type=jnp.float32)


def _tc_a(emb, W1, b1, W2, dinv_c, cc_c):
    grid = (NP // BM,)
    cst = lambda i: (0, 0)
    row = lambda i: (i, 0)
    return pl.pallas_call(
        _tc_a_body,
        grid=grid,
        in_specs=[
            pl.BlockSpec((1, D), cst),
            pl.BlockSpec((D, D), cst),
            pl.BlockSpec((1, D), cst),
            pl.BlockSpec((D, D), cst),
            pl.BlockSpec((BM, 1), row),
            pl.BlockSpec((BM, 1), row),
        ],
        out_specs=[pl.BlockSpec((BM, D), row), pl.BlockSpec((BM, D), row)],
        out_shape=[jax.ShapeDtypeStruct((NP, D), jnp.float32),
                   jax.ShapeDtypeStruct((NP, D), jnp.float32)],
    )(emb, W1, b1, W2, dinv_c, cc_c)


def _tc_b_body(p_r, g2_r, h1_r, dinv_r, emb_r, w3_r, b2_r, h2_r, g3_r):
    agg = dinv_r[...] * (p_r[0] + p_r[1] + g2_r[...])
    h2 = _elu(agg + b2_r[...])
    h2_r[...] = h2
    x3 = emb_r[...] + h1_r[...] + h2
    g3_r[...] = dinv_r[...] * jnp.dot(x3, w3_r[...],
                                      preferred_element_type=jnp.float32)


def _tc_b(p, g2, h1, dinv_c, emb, W3, b2):
    grid = (NP // BM,)
    cst = lambda i: (0, 0)
    row = lambda i: (i, 0)
    return pl.pallas_call(
        _tc_b_body,
        grid=grid,
        in_specs=[
            pl.BlockSpec((2, BM, D), lambda i: (0, i, 0)),
            pl.BlockSpec((BM, D), row),
            pl.BlockSpec((BM, D), row),
            pl.BlockSpec((BM, 1), row),
            pl.BlockSpec((1, D), cst),
            pl.BlockSpec((D, D), cst),
            pl.BlockSpec((1, D), cst),
        ],
        out_specs=[pl.BlockSpec((BM, D), row), pl.BlockSpec((BM, D), row)],
        out_shape=[jax.ShapeDtypeStruct((NP, D), jnp.float32),
                   jax.ShapeDtypeStruct((NP, D), jnp.float32)],
    )(p, g2, h1, dinv_c, emb, W3, b2)


def _tc_c_body(p_r, g3_r, h1_r, h2_r, dinv_r, emb_r, b3_r, lw_r, lb_r, pw_r,
               pb_r, batch_r, out_r, pooled):
    i = pl.program_id(0)
    h3 = _elu(dinv_r[...] * (p_r[0] + p_r[1] + g3_r[...]) + b3_r[...])
    xf = emb_r[...] + h1_r[...] + h2_r[...] + h3
    hf = _elu(jnp.dot(xf, lw_r[...], preferred_element_type=jnp.float32)
              + lb_r[...])
    rid = i * BM + lax.broadcasted_iota(jnp.int32, (BM, 1), 0)
    hf = jnp.where(rid < N, hf, 0.0)
    oh = (batch_r[...] == lax.broadcasted_iota(jnp.int32, (BM, G), 1)
          ).astype(jnp.float32)
    contrib = lax.dot_general(oh, hf, (((0,), (0,)), ((), ())),
                              preferred_element_type=jnp.float32)

    @pl.when(i == 0)
    def _():
        pooled[...] = jnp.zeros((G, D), jnp.float32)

    pooled[...] += contrib
    out_r[...] = (jnp.dot(pooled[...], pw_r[...],
                          preferred_element_type=jnp.float32)
                  + pb_r[...]) * 0.1


def _tc_c(p, g3, h1, h2, dinv_c, emb, b3, last_W, last_b, pred_W, pred_b,
          batch_c):
    grid = (NP // BM,)
    cst = lambda i: (0, 0)
    row = lambda i: (i, 0)
    return pl.pallas_call(
        _tc_c_body,
        grid=grid,
        in_specs=[
            pl.BlockSpec((2, BM, D), lambda i: (0, i, 0)),
            pl.BlockSpec((BM, D), row),
            pl.BlockSpec((BM, D), row),
            pl.BlockSpec((BM, D), row),
            pl.BlockSpec((BM, 1), row),
            pl.BlockSpec((1, D), cst),
            pl.BlockSpec((1, D), cst),
            pl.BlockSpec((D, D), cst),
            pl.BlockSpec((1, D), cst),
            pl.BlockSpec((D, OUT), cst),
            pl.BlockSpec((1, OUT), cst),
            pl.BlockSpec((BM, 1), row),
        ],
        out_specs=pl.BlockSpec((G, OUT), cst),
        out_shape=jax.ShapeDtypeStruct((G, OUT), jnp.float32),
        scratch_shapes=[pltpu.VMEM((G, D), jnp.float32)],
    )(p, g3, h1, h2, dinv_c, emb, b3, last_W, last_b, pred_W, pred_b, batch_c)


def kernel(x, edge_index, batch, edge_attr, emb, W1, b1, W2, b2, W3, b3,
           last_W, last_b, pred_W, pred_b):
    src = edge_index[0]
    dst = edge_index[1]
    srcp = jnp.concatenate(
        [src, jnp.zeros((EPAD - E,), jnp.int32)]).reshape(TOTC, KC)
    dstp = jnp.concatenate(
        [dst, jnp.full((EPAD - E,), N, jnp.int32)]).reshape(TOTC, KC)
    batch_c = jnp.concatenate(
        [batch, jnp.full((NP - N,), G, jnp.int32)]).reshape(NP, 1)

    dinv, cc = _sc_scalar(srcp, dstp)
    dinv_c = dinv.reshape(NP, 1)
    cc_c = cc.reshape(NP, 1)

    h1, g2 = _tc_a(emb, W1, b1.reshape(1, D), W2, dinv_c, cc_c)
    p2 = _sc_agg(g2, srcp, dstp)
    h2, g3 = _tc_b(p2, g2, h1, dinv_c, emb, W3, b2.reshape(1, D))
    p3 = _sc_agg(g3, srcp, dstp)
    out = _tc_c(p3, g3, h1, h2, dinv_c, emb, b3.reshape(1, D), last_W,
                last_b.reshape(1, D), pred_W, pred_b.reshape(1, OUT), batch_c)
    return out


# speedup vs baseline: 2.1315x; 1.0009x over previous
import functools

import jax
import jax.numpy as jnp
from jax import lax
from jax.experimental import pallas as pl
from jax.experimental.pallas import tpu as pltpu
from jax.experimental.pallas import tpu_sc as plsc

N = 10000
E = 320000
D = 128
G = 128
OUT = 128

NP = 10240
KC = 120
TOTC = 2688
EPAD = TOTC * KC
IR = 5
AC0 = 136
AC1 = 168 - AC0
CH_SC = TOTC // 16
ROWS_T = NP // 16
BM = 2048

_mesh = plsc.VectorSubcoreMesh(core_axis_name="c", subcore_axis_name="s")
_sc_params = pltpu.CompilerParams(needs_layout_passes=False)


def _rsqrt16(x):
    s = 0.5 * (1.0 + x)
    for _ in range(15):
        s = 0.5 * (s + x / s)
    return 1.0 / s


def _zero_fill(buf, nrows):
    def body(i, _):
        for j in range(8):
            buf[i, pl.ds(j * 16, 16)] = jnp.zeros((16,), jnp.float32)
        return 0
    lax.fori_loop(0, nrows, body, 0)


NSEM = 8


def _fire_drain(nchunks, fire):
    full = nchunks // NSEM
    rem = nchunks - full * NSEM

    def grp(g, _):
        base = g * NSEM
        ds_ = [fire(base + k, k) for k in range(NSEM)]
        for dsc in ds_:
            dsc.wait()
        return 0
    lax.fori_loop(0, full, grp, 0)
    ds_ = [fire(full * NSEM + k, k) for k in range(rem)]
    for dsc in ds_:
        dsc.wait()


@functools.partial(
    pl.kernel,
    out_type=(jax.ShapeDtypeStruct((NP,), jnp.float32),
              jax.ShapeDtypeStruct((NP,), jnp.float32)),
    mesh=_mesh,
    compiler_params=_sc_params,
    scratch_types=dict(
        deg_acc=pltpu.VMEM_SHARED((NP,), jnp.float32),
        cs_acc=pltpu.VMEM_SHARED((NP,), jnp.float32),
        dinv_sh=pltpu.VMEM_SHARED((NP,), jnp.float32),
        onesv=pltpu.VMEM((KC,), jnp.float32),
        srcv=pltpu.VMEM((CH_SC, KC), jnp.int32),
        dstv=pltpu.VMEM((CH_SC, KC), jnp.int32),
        valv=pltpu.VMEM((CH_SC, KC), jnp.float32),
        dv=pltpu.VMEM((NP,), jnp.float32),
        dslice=pltpu.VMEM((ROWS_T,), jnp.float32),
        csv=pltpu.VMEM((ROWS_T,), jnp.float32),
        sems=pltpu.SemaphoreType.DMA((NSEM,)),
    ),
)
def _sc_scalar(src_hbm, dst_hbm, dinv_out, cc_out, *, deg_acc, cs_acc,
               dinv_sh, onesv, srcv, dstv, valv, dv, dslice, csv, sems):
    c = lax.axis_index("c")
    s = lax.axis_index("s")

    @pl.when(c == 0)
    def _():
        r0 = s * ROWS_T
        def zb(i, _):
            dslice[pl.ds(i * 16, 16)] = jnp.zeros((16,), jnp.float32)
            return 0
        lax.fori_loop(0, ROWS_T // 16, zb, 0)
        pltpu.sync_copy(dslice, deg_acc.at[pl.ds(r0, ROWS_T)])
        pltpu.sync_copy(dslice, cs_acc.at[pl.ds(r0, ROWS_T)])

        def ob(i, _):
            onesv[pl.ds(i * 16, 16)] = jnp.ones((16,), jnp.float32)
            return 0
        lax.fori_loop(0, KC // 16, ob, 0)
        if KC % 16:
            onesv[pl.ds(KC - 16, 16)] = jnp.ones((16,), jnp.float32)
        plsc.subcore_barrier()

        pltpu.sync_copy(dst_hbm.at[pl.ds(s * CH_SC, CH_SC)], dstv)

        def fire_deg(i, k):
            return pltpu.async_copy(
                onesv, deg_acc.at[dstv.at[i]], sems.at[k], add=True)
        _fire_drain(CH_SC, fire_deg)
        plsc.subcore_barrier()

        pltpu.sync_copy(deg_acc.at[pl.ds(r0, ROWS_T)], csv)

        def rb(i, _):
            x = csv[pl.ds(i * 16, 16)] + 1.0
            dslice[pl.ds(i * 16, 16)] = _rsqrt16(x)
            return 0
        lax.fori_loop(0, ROWS_T // 16, rb, 0)
        pltpu.sync_copy(dslice, dinv_sh.at[pl.ds(r0, ROWS_T)])
        plsc.subcore_barrier()

        pltpu.sync_copy(dinv_sh, dv)
        pltpu.sync_copy(src_hbm.at[pl.ds(s * CH_SC, CH_SC)], srcv)

        def gb(i, _):
            offs = [j * 16 for j in range(KC // 16)]
            if KC % 16:
                offs.append(KC - 16)
            for o in offs:
                idx = srcv[i, pl.ds(o, 16)]
                valv[i, pl.ds(o, 16)] = plsc.load_gather(dv, [idx])
            return 0
        lax.fori_loop(0, CH_SC, gb, 0)

        def fire_cs(i, k):
            return pltpu.async_copy(
                valv.at[i], cs_acc.at[dstv.at[i]], sems.at[k], add=True)
        _fire_drain(CH_SC, fire_cs)
        plsc.subcore_barrier()

        pltpu.sync_copy(cs_acc.at[pl.ds(r0, ROWS_T)], csv)

        def cb(i, _):
            dvv = dslice[pl.ds(i * 16, 16)]
            csv[pl.ds(i * 16, 16)] = dvv * (csv[pl.ds(i * 16, 16)] + dvv)
            return 0
        lax.fori_loop(0, ROWS_T // 16, cb, 0)
        pltpu.sync_copy(dslice, dinv_out.at[pl.ds(r0, ROWS_T)])
        pltpu.sync_copy(csv, cc_out.at[pl.ds(r0, ROWS_T)])


RING = 3
NAGG = 10112
ROWS_A = NAGG // 16


@functools.partial(
    pl.kernel,
    out_type=jax.ShapeDtypeStruct((2, NP, D), jnp.float32),
    mesh=_mesh,
    compiler_params=_sc_params,
    scratch_types=dict(
        acc=pltpu.VMEM_SHARED((NAGG, D), jnp.float32),
        sidx=pltpu.VMEM((IR, KC), jnp.int32),
        didx=pltpu.VMEM((IR, KC), jnp.int32),
        rowbuf=pltpu.VMEM((RING, KC, D), jnp.float32),
        gsems=pltpu.SemaphoreType.DMA((RING,)),
        ssems=pltpu.SemaphoreType.DMA((RING,)),
        isems=pltpu.SemaphoreType.DMA((IR,)),
    ),
)
def _sc_agg(gsc_hbm, src_hbm, dst_hbm, out_hbm, *, acc, sidx, didx, rowbuf,
            gsems, ssems, isems):
    c = lax.axis_index("c")
    s = lax.axis_index("s")
    r0 = s * ROWS_A
    nc = jnp.where(c == 0, AC0, AC1)
    base = jnp.where(c == 0, s * AC0, 16 * AC0 + s * AC1)

    _zero_fill(rowbuf.at[0], KC)
    zsems = [gsems.at[k] for k in range(RING)] + [ssems.at[k] for k in range(RING)]
    rem_rows = ROWS_A - (ROWS_A // KC) * KC
    zd = [pltpu.async_copy(rowbuf.at[0], acc.at[pl.ds(r0 + k * KC, KC)],
                           zsems[k]) for k in range(ROWS_A // KC)]
    zd.append(pltpu.async_copy(rowbuf.at[0, pl.ds(0, rem_rows)],
                               acc.at[pl.ds(r0 + (ROWS_A // KC) * KC,
                                            rem_rows)],
                               zsems[ROWS_A // KC]))
    for dsc in zd:
        dsc.wait()
    plsc.subcore_barrier()

    pltpu.sync_copy(src_hbm.at[base], sidx.at[0])
    pltpu.sync_copy(dst_hbm.at[base], didx.at[0])
    for k in (1, 2):
        pltpu.async_copy(src_hbm.at[base + k], sidx.at[k], isems.at[k])
        pltpu.async_copy(dst_hbm.at[base + k], didx.at[k], isems.at[k])
    pltpu.async_copy(gsc_hbm.at[sidx.at[0]], rowbuf.at[0], gsems.at[0])

    def it(i, _):
        b = lax.rem(i, RING)
        nxt = lax.rem(i + 1, RING)
        m1 = lax.rem(i + 1, IR)
        m3 = lax.rem(i + 3, IR)

        @pl.when(jnp.logical_and(i >= 2, i <= nc - 2))
        def _():
            pltpu.make_async_copy(rowbuf.at[nxt], acc.at[didx.at[0]],
                                  ssems.at[nxt]).wait()

        @pl.when(i <= nc - 4)
        def _():
            pltpu.async_copy(src_hbm.at[base + i + 3], sidx.at[m3],
                             isems.at[m3])
            pltpu.async_copy(dst_hbm.at[base + i + 3], didx.at[m3],
                             isems.at[m3])

        @pl.when(i <= nc - 2)
        def _():
            pltpu.make_async_copy(src_hbm.at[base + i + 1], sidx.at[m1],
                                  isems.at[m1]).wait()
            pltpu.make_async_copy(dst_hbm.at[base + i + 1], didx.at[m1],
                                  isems.at[m1]).wait()
            pltpu.async_copy(gsc_hbm.at[sidx.at[m1]], rowbuf.at[nxt],
                             gsems.at[nxt])

        mi = lax.rem(i, IR)
        pltpu.make_async_copy(gsc_hbm.at[sidx.at[mi]], rowbuf.at[b],
                              gsems.at[b]).wait()
        pltpu.async_copy(rowbuf.at[b], acc.at[didx.at[mi]], ssems.at[b],
                         add=True)
        return 0

    lax.fori_loop(0, nc, it, 0)
    for v in range(RING):
        pltpu.make_async_copy(rowbuf.at[v], acc.at[didx.at[0]],
                              ssems.at[v]).wait()

    plsc.subcore_barrier()
    pltpu.sync_copy(acc.at[pl.ds(r0, ROWS_A)],
                    out_hbm.at[c, pl.ds(r0, ROWS_A)])


def _elu(z):
    return jnp.where(z > 0, z, jnp.exp(z) - 1.0)


def _tc_a_body(emb_r, w1_r, b1_r, w2_r, dinv_r, cc_r, h1_r, g2_r):
    e = emb_r[...]
    r1 = jnp.dot(e, w1_r[...], preferred_element_type=jnp.float32)
    h1 = _elu(cc_r[...] * r1 + b1_r[...])
    h1_r[...] = h1
    x2 = h1 + e
    g2_r[...] = dinv_r[...] * jnp.dot(x2, w2_r[...],
                                      preferred_element_type=jnp.float32)


def _tc_a(emb, W1, b1, W2, dinv_c, cc_c):
    grid = (NP // BM,)
    cst = lambda i: (0, 0)
    row = lambda i: (i, 0)
    return pl.pallas_call(
        _tc_a_body,
        grid=grid,
        in_specs=[
            pl.BlockSpec((1, D), cst),
            pl.BlockSpec((D, D), cst),
            pl.BlockSpec((1, D), cst),
            pl.BlockSpec((D, D), cst),
            pl.BlockSpec((BM, 1), row),
            pl.BlockSpec((BM, 1), row),
        ],
        out_specs=[pl.BlockSpec((BM, D), row), pl.BlockSpec((BM, D), row)],
        out_shape=[jax.ShapeDtypeStruct((NP, D), jnp.float32),
                   jax.ShapeDtypeStruct((NP, D), jnp.float32)],
    )(emb, W1, b1, W2, dinv_c, cc_c)


def _tc_b_body(p_r, g2_r, h1_r, dinv_r, emb_r, w3_r, b2_r, h2_r, g3_r):
    agg = dinv_r[...] * (p_r[0] + p_r[1] + g2_r[...])
    h2 = _elu(agg + b2_r[...])
    h2_r[...] = h2
    x3 = emb_r[...] + h1_r[...] + h2
    g3_r[...] = dinv_r[...] * jnp.dot(x3, w3_r[...],
                                      preferred_element_type=jnp.float32)


def _tc_b(p, g2, h1, dinv_c, emb, W3, b2):
    grid = (NP // BM,)
    cst = lambda i: (0, 0)
    row = lambda i: (i, 0)
    return pl.pallas_call(
        _tc_b_body,
        grid=grid,
        in_specs=[
            pl.BlockSpec((2, BM, D), lambda i: (0, i, 0)),
            pl.BlockSpec((BM, D), row),
            pl.BlockSpec((BM, D), row),
            pl.BlockSpec((BM, 1), row),
            pl.BlockSpec((1, D), cst),
            pl.BlockSpec((D, D), cst),
            pl.BlockSpec((1, D), cst),
        ],
        out_specs=[pl.BlockSpec((BM, D), row), pl.BlockSpec((BM, D), row)],
        out_shape=[jax.ShapeDtypeStruct((NP, D), jnp.float32),
                   jax.ShapeDtypeStruct((NP, D), jnp.float32)],
    )(p, g2, h1, dinv_c, emb, W3, b2)


def _tc_c_body(p_r, g3_r, h1_r, h2_r, dinv_r, emb_r, b3_r, lw_r, lb_r, pw_r,
               pb_r, batch_r, out_r, pooled):
    i = pl.program_id(0)
    h3 = _elu(dinv_r[...] * (p_r[0] + p_r[1] + g3_r[...]) + b3_r[...])
    xf = emb_r[...] + h1_r[...] + h2_r[...] + h3
    hf = _elu(jnp.dot(xf, lw_r[...], preferred_element_type=jnp.float32)
              + lb_r[...])
    rid = i * BM + lax.broadcasted_iota(jnp.int32, (BM, 1), 0)
    hf = jnp.where(rid < N, hf, 0.0)
    oh = (batch_r[...] == lax.broadcasted_iota(jnp.int32, (BM, G), 1)
          ).astype(jnp.float32)
    contrib = lax.dot_general(oh, hf, (((0,), (0,)), ((), ())),
                              preferred_element_type=jnp.float32)

    @pl.when(i == 0)
    def _():
        pooled[...] = jnp.zeros((G, D), jnp.float32)

    pooled[...] += contrib
    out_r[...] = (jnp.dot(pooled[...], pw_r[...],
                          preferred_element_type=jnp.float32)
                  + pb_r[...]) * 0.1


def _tc_c(p, g3, h1, h2, dinv_c, emb, b3, last_W, last_b, pred_W, pred_b,
          batch_c):
    grid = (NP // BM,)
    cst = lambda i: (0, 0)
    row = lambda i: (i, 0)
    return pl.pallas_call(
        _tc_c_body,
        grid=grid,
        in_specs=[
            pl.BlockSpec((2, BM, D), lambda i: (0, i, 0)),
            pl.BlockSpec((BM, D), row),
            pl.BlockSpec((BM, D), row),
            pl.BlockSpec((BM, D), row),
            pl.BlockSpec((BM, 1), row),
            pl.BlockSpec((1, D), cst),
            pl.BlockSpec((1, D), cst),
            pl.BlockSpec((D, D), cst),
            pl.BlockSpec((1, D), cst),
            pl.BlockSpec((D, OUT), cst),
            pl.BlockSpec((1, OUT), cst),
            pl.BlockSpec((BM, 1), row),
        ],
        out_specs=pl.BlockSpec((G, OUT), cst),
        out_shape=jax.ShapeDtypeStruct((G, OUT), jnp.float32),
        scratch_shapes=[pltpu.VMEM((G, D), jnp.float32)],
    )(p, g3, h1, h2, dinv_c, emb, b3, last_W, last_b, pred_W, pred_b, batch_c)


def kernel(x, edge_index, batch, edge_attr, emb, W1, b1, W2, b2, W3, b3,
           last_W, last_b, pred_W, pred_b):
    src = edge_index[0]
    dst = edge_index[1]
    srcp = jnp.concatenate(
        [src, jnp.zeros((EPAD - E,), jnp.int32)]).reshape(TOTC, KC)
    dstp = jnp.concatenate(
        [dst, jnp.full((EPAD - E,), N, jnp.int32)]).reshape(TOTC, KC)
    batch_c = jnp.concatenate(
        [batch, jnp.full((NP - N,), G, jnp.int32)]).reshape(NP, 1)

    dinv, cc = _sc_scalar(srcp, dstp)
    dinv_c = dinv.reshape(NP, 1)
    cc_c = cc.reshape(NP, 1)

    h1, g2 = _tc_a(emb, W1, b1.reshape(1, D), W2, dinv_c, cc_c)
    p2 = _sc_agg(g2, srcp, dstp)
    h2, g3 = _tc_b(p2, g2, h1, dinv_c, emb, W3, b2.reshape(1, D))
    p3 = _sc_agg(g3, srcp, dstp)
    out = _tc_c(p3, g3, h1, h2, dinv_c, emb, b3.reshape(1, D), last_W,
                last_b.reshape(1, D), pred_W, pred_b.reshape(1, OUT), batch_c)
    return out
